# Initial kernel scaffold; baseline (speedup 1.0000x reference)
import jax
import jax.numpy as jnp
from jax.experimental import pallas as pl


def kernel(x, edge_index, W1, b1, W2, b2, W_out, b_out):
    raise NotImplementedError("write your pallas kernel here")


# speedup vs baseline: 3.0561x; 3.0561x over previous
import functools

import jax
import jax.numpy as jnp
from jax import lax
from jax.experimental import pallas as pl
from jax.experimental.pallas import tpu as pltpu
from jax.experimental.pallas import tpu_sc as plsc

N = 10000
E = 160000
D_IN = 256
D_H = 1024
D_OUT = 128

NCORES = 2
NTILES = 16
K = 128
EPAD = 163840
CHUNKS_PER_TILE = EPAD // (K * NTILES)
NPAD_ROWS = 48
SHARE_ROWS = 632
AGG_OUT_ROWS = NTILES * SHARE_ROWS
NP1 = AGG_OUT_ROWS + NPAD_ROWS
DEG_PAD = 10240
DEG_SHARE = DEG_PAD // NTILES

BN = 1000
NB_N = N // BN


def _hist_body(srcp, dstp, dego, degi, deg_sp, idx_v, ones_v, zer_v):
    cid = lax.axis_index("c")
    sid = lax.axis_index("s")
    for i in range(8):
        ones_v[pl.ds(i * 16, 16)] = jnp.ones((16,), jnp.float32)

    def zfill(i, c):
        zer_v[pl.ds(i * 16, 16)] = jnp.zeros((16,), jnp.float32)
        return c
    lax.fori_loop(0, DEG_SHARE // 16, zfill, 0)

    base = pl.multiple_of(sid * DEG_SHARE, 8)
    pltpu.sync_copy(zer_v, deg_sp.at[pl.ds(base, DEG_SHARE)])
    plsc.subcore_barrier()

    total_chunks = E // K
    nbase, nrem = divmod(total_chunks, NTILES)
    n_j = nbase + jnp.where(sid < nrem, 1, 0)

    def accumulate(edge_ref):
        def body(j, c):
            e0 = pl.multiple_of((sid + NTILES * j) * K, 8)
            pltpu.sync_copy(edge_ref.at[pl.ds(e0, K)], idx_v)
            pltpu.sync_copy(ones_v, deg_sp.at[idx_v], add=True)
            return c
        lax.fori_loop(0, n_j, body, 0)

    @pl.when(cid == 0)
    def _():
        accumulate(srcp)

    @pl.when(cid == 1)
    def _():
        accumulate(dstp)

    plsc.subcore_barrier()

    @pl.when(cid == 0)
    def _():
        pltpu.sync_copy(deg_sp.at[pl.ds(base, DEG_SHARE)],
                        dego.at[pl.ds(base, DEG_SHARE)])

    @pl.when(cid == 1)
    def _():
        pltpu.sync_copy(deg_sp.at[pl.ds(base, DEG_SHARE)],
                        degi.at[pl.ds(base, DEG_SHARE)])


def _hist(src_p, dst_p):
    mesh = plsc.VectorSubcoreMesh(core_axis_name="c", subcore_axis_name="s")
    return pl.kernel(
        _hist_body,
        out_type=(pltpu.HBM((DEG_PAD,), jnp.float32),
                  pltpu.HBM((DEG_PAD,), jnp.float32)),
        mesh=mesh,
        scratch_types=[
            pltpu.VMEM_SHARED((DEG_PAD,), jnp.float32),
            pltpu.VMEM((K,), jnp.int32),
            pltpu.VMEM((K,), jnp.float32),
            pltpu.VMEM((DEG_SHARE,), jnp.float32),
        ],
    )(src_p, dst_p)


def _agg_body(nblk_per_core, xb, srcp, dstp, zeros_hbm, out,
              agg_sp, rows_v, sidx_v, didx_v, sem):
    cid = lax.axis_index("c")
    sid = lax.axis_index("s")

    rbase = sid * SHARE_ROWS
    for b in range(nblk_per_core):
        bg = cid * nblk_per_core + b
        pltpu.sync_copy(zeros_hbm, agg_sp.at[pl.ds(rbase, SHARE_ROWS)])
        plsc.subcore_barrier()

        def body(j, c):
            e0 = pl.multiple_of((sid * CHUNKS_PER_TILE + j) * K, 8)
            pltpu.sync_copy(srcp.at[pl.ds(e0, K)], sidx_v)
            pltpu.sync_copy(dstp.at[pl.ds(e0, K)], didx_v)
            pltpu.async_copy(xb.at[bg].at[sidx_v], rows_v, sem).wait()
            pltpu.sync_copy(rows_v, agg_sp.at[didx_v], add=True)
            return c
        lax.fori_loop(0, CHUNKS_PER_TILE, body, 0)
        plsc.subcore_barrier()

        obase = pl.multiple_of(bg * AGG_OUT_ROWS + rbase, 8)
        pltpu.sync_copy(agg_sp.at[pl.ds(rbase, SHARE_ROWS)],
                        out.at[pl.ds(obase, SHARE_ROWS)])
        plsc.subcore_barrier()


def _agg(xb, src_p, dst_p, nblocks):
    mesh = plsc.VectorSubcoreMesh(core_axis_name="c", subcore_axis_name="s")
    return pl.kernel(
        functools.partial(_agg_body, nblocks // NCORES),
        out_type=pltpu.HBM((nblocks * AGG_OUT_ROWS, 128), jnp.float32),
        mesh=mesh,
        scratch_types=[
            pltpu.VMEM_SHARED((NP1, 128), jnp.float32),
            pltpu.VMEM((K, 128), jnp.float32),
            pltpu.VMEM((K,), jnp.int32),
            pltpu.VMEM((K,), jnp.int32),
            pltpu.SemaphoreType.DMA,
        ],
    )(xb, src_p, dst_p,
      jnp.zeros((SHARE_ROWS, 128), jnp.float32)).reshape(nblocks, AGG_OUT_ROWS, 128)


def _prep_body(x_ref, dego_ref, xs_ref):
    s = lax.rsqrt(jnp.maximum(dego_ref[...], 1.0))
    xs_ref[0] = x_ref[...] * s


def _prep(x, dego2):
    nb_f = D_IN // 128
    return pl.pallas_call(
        _prep_body,
        grid=(nb_f, NB_N),
        in_specs=[
            pl.BlockSpec((BN, 128), lambda b, n: (n, b)),
            pl.BlockSpec((BN, 1), lambda b, n: (n, 0)),
        ],
        out_specs=pl.BlockSpec((1, BN, 128), lambda b, n: (b, n, 0)),
        out_shape=jax.ShapeDtypeStruct((nb_f, N, 128), jnp.float32),
    )(x, dego2)


def _mm1_body(nk, agg_ref, degi_ref, dego_ref, w_ref, b_ref, out_ref, acc_ref):
    k = pl.program_id(2)

    @pl.when(k == 0)
    def _():
        acc_ref[...] = jnp.zeros_like(acc_ref)

    s_in = lax.rsqrt(jnp.maximum(degi_ref[...], 1.0))
    acc_ref[...] += jnp.dot(agg_ref[0] * s_in, w_ref[...],
                            preferred_element_type=jnp.float32)

    @pl.when(k == nk - 1)
    def _():
        s_out = lax.rsqrt(jnp.maximum(dego_ref[...], 1.0))
        out_ref[0] = jnp.maximum(acc_ref[...] + b_ref[...], 0.0) * s_out


def _mm1(agg1, degi2, dego2, W1, b1r):
    nk = D_IN // 128
    no = D_H // 128
    return pl.pallas_call(
        functools.partial(_mm1_body, nk),
        grid=(no, NB_N, nk),
        in_specs=[
            pl.BlockSpec((1, BN, 128), lambda o, n, k: (k, n, 0)),
            pl.BlockSpec((BN, 1), lambda o, n, k: (n, 0)),
            pl.BlockSpec((BN, 1), lambda o, n, k: (n, 0)),
            pl.BlockSpec((128, 128), lambda o, n, k: (k, o)),
            pl.BlockSpec((1, 128), lambda o, n, k: (0, o)),
        ],
        out_specs=pl.BlockSpec((1, BN, 128), lambda o, n, k: (o, n, 0)),
        out_shape=jax.ShapeDtypeStruct((no, N, 128), jnp.float32),
        scratch_shapes=[pltpu.VMEM((BN, 128), jnp.float32)],
    )(agg1, degi2, dego2, W1, b1r)


def _mm2_body(agg_ref, degi_ref, w2_ref, b2_ref, wo_ref, bo_ref, out_ref,
              acc_ref, csum_ref):
    o = pl.program_id(0)
    n = pl.program_id(1)
    k = pl.program_id(2)
    nk = pl.num_programs(2)

    @pl.when(k == 0)
    def _():
        acc_ref[...] = jnp.zeros_like(acc_ref)

    s_in = lax.rsqrt(jnp.maximum(degi_ref[...], 1.0))
    acc_ref[...] += jnp.dot(agg_ref[0] * s_in, w2_ref[...],
                            preferred_element_type=jnp.float32)

    @pl.when(k == nk - 1)
    def _():
        h = jnp.maximum(acc_ref[...] + b2_ref[...], 0.0)
        part = jnp.sum(h, axis=0, keepdims=True)

        @pl.when(n == 0)
        def _():
            csum_ref[...] = part

        @pl.when(n > 0)
        def _():
            csum_ref[...] += part

        @pl.when(n == NB_N - 1)
        def _():
            contrib = jnp.dot(csum_ref[...] * (1.0 / N), wo_ref[...],
                              preferred_element_type=jnp.float32)

            @pl.when(o == 0)
            def _():
                out_ref[...] = contrib + bo_ref[...]

            @pl.when(o > 0)
            def _():
                out_ref[...] += contrib


def _mm2(agg2, degi2, W2, b2r, W_out, bor):
    nk = D_H // 128
    no = D_H // 128
    return pl.pallas_call(
        _mm2_body,
        grid=(no, NB_N, nk),
        in_specs=[
            pl.BlockSpec((1, BN, 128), lambda o, n, k: (k, n, 0)),
            pl.BlockSpec((BN, 1), lambda o, n, k: (n, 0)),
            pl.BlockSpec((128, 128), lambda o, n, k: (k, o)),
            pl.BlockSpec((1, 128), lambda o, n, k: (0, o)),
            pl.BlockSpec((128, D_OUT), lambda o, n, k: (o, 0)),
            pl.BlockSpec((1, D_OUT), lambda o, n, k: (0, 0)),
        ],
        out_specs=pl.BlockSpec((1, D_OUT), lambda o, n, k: (0, 0)),
        out_shape=jax.ShapeDtypeStruct((1, D_OUT), jnp.float32),
        scratch_shapes=[pltpu.VMEM((BN, 128), jnp.float32),
                        pltpu.VMEM((1, 128), jnp.float32)],
    )(agg2, degi2, W2, b2r, W_out, bor)


def kernel(x, edge_index, W1, b1, W2, b2, W_out, b_out):
    src = edge_index[0]
    dst = edge_index[1]
    pad_ids = jnp.arange(EPAD - E, dtype=jnp.int32) % NPAD_ROWS
    src_p = jnp.concatenate([src, pad_ids])
    dst_p = jnp.concatenate([dst, AGG_OUT_ROWS + pad_ids])

    dego, degi = _hist(src, dst)
    dego2 = dego[:N].reshape(N, 1)
    degi2 = degi[:N].reshape(N, 1)

    xs = _prep(x, dego2)
    agg1 = _agg(xs, src_p, dst_p, D_IN // 128)
    h1s = _mm1(agg1, degi2, dego2, W1, b1.reshape(1, -1))
    agg2 = _agg(h1s, src_p, dst_p, D_H // 128)
    return _mm2(agg2, degi2, W2, b2.reshape(1, -1), W_out, b_out.reshape(1, -1))


# speedup vs baseline: 4.0592x; 1.3282x over previous
import functools

import jax
import jax.numpy as jnp
from jax import lax
from jax.experimental import pallas as pl
from jax.experimental.pallas import tpu as pltpu
from jax.experimental.pallas import tpu_sc as plsc

N = 10000
E = 160000
D_IN = 256
D_H = 1024
D_OUT = 128

NCORES = 2
NTILES = 16
K = 128
EPAD = 163840
CHUNKS_PER_TILE = EPAD // (K * NTILES)
NPAD_ROWS = 48
SHARE_ROWS = 632
AGG_OUT_ROWS = NTILES * SHARE_ROWS
NP1 = AGG_OUT_ROWS + NPAD_ROWS
DEG_PAD = 10240
DEG_SHARE = DEG_PAD // NTILES

BN = 1000
NB_N = N // BN


def _hist_body(srcp, dstp, dego, degi, deg_sp, idx_v, ones_v, zer_v):
    cid = lax.axis_index("c")
    sid = lax.axis_index("s")
    for i in range(8):
        ones_v[pl.ds(i * 16, 16)] = jnp.ones((16,), jnp.float32)

    def zfill(i, c):
        zer_v[pl.ds(i * 16, 16)] = jnp.zeros((16,), jnp.float32)
        return c
    lax.fori_loop(0, DEG_SHARE // 16, zfill, 0)

    base = pl.multiple_of(sid * DEG_SHARE, 8)
    pltpu.sync_copy(zer_v, deg_sp.at[pl.ds(base, DEG_SHARE)])
    plsc.subcore_barrier()

    total_chunks = E // K
    nbase, nrem = divmod(total_chunks, NTILES)
    n_j = nbase + jnp.where(sid < nrem, 1, 0)

    def accumulate(edge_ref):
        def body(j, c):
            e0 = pl.multiple_of((sid + NTILES * j) * K, 8)
            pltpu.sync_copy(edge_ref.at[pl.ds(e0, K)], idx_v)
            pltpu.sync_copy(ones_v, deg_sp.at[idx_v], add=True)
            return c
        lax.fori_loop(0, n_j, body, 0)

    @pl.when(cid == 0)
    def _():
        accumulate(srcp)

    @pl.when(cid == 1)
    def _():
        accumulate(dstp)

    plsc.subcore_barrier()

    @pl.when(cid == 0)
    def _():
        pltpu.sync_copy(deg_sp.at[pl.ds(base, DEG_SHARE)],
                        dego.at[pl.ds(base, DEG_SHARE)])

    @pl.when(cid == 1)
    def _():
        pltpu.sync_copy(deg_sp.at[pl.ds(base, DEG_SHARE)],
                        degi.at[pl.ds(base, DEG_SHARE)])


def _hist(src_p, dst_p):
    mesh = plsc.VectorSubcoreMesh(core_axis_name="c", subcore_axis_name="s")
    return pl.kernel(
        _hist_body,
        out_type=(pltpu.HBM((DEG_PAD,), jnp.float32),
                  pltpu.HBM((DEG_PAD,), jnp.float32)),
        mesh=mesh,
        scratch_types=[
            pltpu.VMEM_SHARED((DEG_PAD,), jnp.float32),
            pltpu.VMEM((K,), jnp.int32),
            pltpu.VMEM((K,), jnp.float32),
            pltpu.VMEM((DEG_SHARE,), jnp.float32),
        ],
    )(src_p, dst_p)


_EDGES_PER_TILE = EPAD // NTILES


def _unpack_chunk(pk_v, j, sidx, didx):
    for t in range(K // 16):
        pk = pk_v[pl.ds(j * K + t * 16, 16)]
        didx[pl.ds(t * 16, 16)] = lax.shift_right_logical(pk, 14)
        sidx[pl.ds(t * 16, 16)] = lax.bitwise_and(pk, (1 << 14) - 1)


def _agg_body(nblk_per_core, xb, packed, zeros_hbm, out,
              agg_sp, pk_v, rows0, rows1, sidx0, didx0, sidx1, didx1,
              g0s, g1s, s0s, s1s):
    cid = lax.axis_index("c")
    sid = lax.axis_index("s")

    ebase = pl.multiple_of(sid * _EDGES_PER_TILE, 8)
    pltpu.sync_copy(packed.at[pl.ds(ebase, _EDGES_PER_TILE)], pk_v)

    rbase = sid * SHARE_ROWS
    for b in range(nblk_per_core):
        bg = cid * nblk_per_core + b
        pltpu.sync_copy(zeros_hbm, agg_sp.at[pl.ds(rbase, SHARE_ROWS)])
        plsc.subcore_barrier()

        xrows = xb.at[bg]

        def pair(p, c):
            _unpack_chunk(pk_v, 2 * p, sidx0, didx0)
            _unpack_chunk(pk_v, 2 * p + 1, sidx1, didx1)
            g0 = pltpu.async_copy(xrows.at[sidx0], rows0, g0s)
            g1 = pltpu.async_copy(xrows.at[sidx1], rows1, g1s)
            g0.wait()
            s0 = pltpu.async_copy(rows0, agg_sp.at[didx0], s0s, add=True)
            g1.wait()
            s1 = pltpu.async_copy(rows1, agg_sp.at[didx1], s1s, add=True)
            s0.wait()
            s1.wait()
            return c
        lax.fori_loop(0, CHUNKS_PER_TILE // 2, pair, 0)
        plsc.subcore_barrier()

        obase = pl.multiple_of(bg * AGG_OUT_ROWS + rbase, 8)
        pltpu.sync_copy(agg_sp.at[pl.ds(rbase, SHARE_ROWS)],
                        out.at[pl.ds(obase, SHARE_ROWS)])
        plsc.subcore_barrier()


def _agg(xb, packed, nblocks):
    mesh = plsc.VectorSubcoreMesh(core_axis_name="c", subcore_axis_name="s")
    return pl.kernel(
        functools.partial(_agg_body, nblocks // NCORES),
        out_type=pltpu.HBM((nblocks * AGG_OUT_ROWS, 128), jnp.float32),
        mesh=mesh,
        scratch_types=[
            pltpu.VMEM_SHARED((NP1, 128), jnp.float32),
            pltpu.VMEM((_EDGES_PER_TILE,), jnp.int32),
            pltpu.VMEM((K, 128), jnp.float32),
            pltpu.VMEM((K, 128), jnp.float32),
            pltpu.VMEM((K,), jnp.int32),
            pltpu.VMEM((K,), jnp.int32),
            pltpu.VMEM((K,), jnp.int32),
            pltpu.VMEM((K,), jnp.int32),
            pltpu.SemaphoreType.DMA,
            pltpu.SemaphoreType.DMA,
            pltpu.SemaphoreType.DMA,
            pltpu.SemaphoreType.DMA,
        ],
    )(xb, packed,
      jnp.zeros((SHARE_ROWS, 128), jnp.float32)).reshape(nblocks, AGG_OUT_ROWS, 128)


def _prep_body(x_ref, dego_ref, xs_ref):
    s = lax.rsqrt(jnp.maximum(dego_ref[...], 1.0))
    xs_ref[0] = x_ref[...] * s


def _prep(x, dego2):
    nb_f = D_IN // 128
    return pl.pallas_call(
        _prep_body,
        grid=(nb_f, NB_N),
        in_specs=[
            pl.BlockSpec((BN, 128), lambda b, n: (n, b)),
            pl.BlockSpec((BN, 1), lambda b, n: (n, 0)),
        ],
        out_specs=pl.BlockSpec((1, BN, 128), lambda b, n: (b, n, 0)),
        out_shape=jax.ShapeDtypeStruct((nb_f, N, 128), jnp.float32),
    )(x, dego2)


def _mm1_body(nk, agg_ref, degi_ref, dego_ref, w_ref, b_ref, out_ref, acc_ref):
    k = pl.program_id(2)

    @pl.when(k == 0)
    def _():
        acc_ref[...] = jnp.zeros_like(acc_ref)

    s_in = lax.rsqrt(jnp.maximum(degi_ref[...], 1.0))
    acc_ref[...] += jnp.dot(agg_ref[0] * s_in, w_ref[...],
                            preferred_element_type=jnp.float32)

    @pl.when(k == nk - 1)
    def _():
        s_out = lax.rsqrt(jnp.maximum(dego_ref[...], 1.0))
        out_ref[0] = jnp.maximum(acc_ref[...] + b_ref[...], 0.0) * s_out


def _mm1(agg1, degi2, dego2, W1, b1r):
    nk = D_IN // 128
    no = D_H // 128
    return pl.pallas_call(
        functools.partial(_mm1_body, nk),
        grid=(no, NB_N, nk),
        in_specs=[
            pl.BlockSpec((1, BN, 128), lambda o, n, k: (k, n, 0)),
            pl.BlockSpec((BN, 1), lambda o, n, k: (n, 0)),
            pl.BlockSpec((BN, 1), lambda o, n, k: (n, 0)),
            pl.BlockSpec((128, 128), lambda o, n, k: (k, o)),
            pl.BlockSpec((1, 128), lambda o, n, k: (0, o)),
        ],
        out_specs=pl.BlockSpec((1, BN, 128), lambda o, n, k: (o, n, 0)),
        out_shape=jax.ShapeDtypeStruct((no, N, 128), jnp.float32),
        scratch_shapes=[pltpu.VMEM((BN, 128), jnp.float32)],
    )(agg1, degi2, dego2, W1, b1r)


def _mm2_body(agg_ref, degi_ref, w2_ref, b2_ref, wo_ref, bo_ref, out_ref,
              acc_ref, csum_ref):
    o = pl.program_id(0)
    n = pl.program_id(1)
    k = pl.program_id(2)
    nk = pl.num_programs(2)

    @pl.when(k == 0)
    def _():
        acc_ref[...] = jnp.zeros_like(acc_ref)

    s_in = lax.rsqrt(jnp.maximum(degi_ref[...], 1.0))
    acc_ref[...] += jnp.dot(agg_ref[0] * s_in, w2_ref[...],
                            preferred_element_type=jnp.float32)

    @pl.when(k == nk - 1)
    def _():
        h = jnp.maximum(acc_ref[...] + b2_ref[...], 0.0)
        part = jnp.sum(h, axis=0, keepdims=True)

        @pl.when(n == 0)
        def _():
            csum_ref[...] = part

        @pl.when(n > 0)
        def _():
            csum_ref[...] += part

        @pl.when(n == NB_N - 1)
        def _():
            contrib = jnp.dot(csum_ref[...] * (1.0 / N), wo_ref[...],
                              preferred_element_type=jnp.float32)

            @pl.when(o == 0)
            def _():
                out_ref[...] = contrib + bo_ref[...]

            @pl.when(o > 0)
            def _():
                out_ref[...] += contrib


def _mm2(agg2, degi2, W2, b2r, W_out, bor):
    nk = D_H // 128
    no = D_H // 128
    return pl.pallas_call(
        _mm2_body,
        grid=(no, NB_N, nk),
        in_specs=[
            pl.BlockSpec((1, BN, 128), lambda o, n, k: (k, n, 0)),
            pl.BlockSpec((BN, 1), lambda o, n, k: (n, 0)),
            pl.BlockSpec((128, 128), lambda o, n, k: (k, o)),
            pl.BlockSpec((1, 128), lambda o, n, k: (0, o)),
            pl.BlockSpec((128, D_OUT), lambda o, n, k: (o, 0)),
            pl.BlockSpec((1, D_OUT), lambda o, n, k: (0, 0)),
        ],
        out_specs=pl.BlockSpec((1, D_OUT), lambda o, n, k: (0, 0)),
        out_shape=jax.ShapeDtypeStruct((1, D_OUT), jnp.float32),
        scratch_shapes=[pltpu.VMEM((BN, 128), jnp.float32),
                        pltpu.VMEM((1, 128), jnp.float32)],
    )(agg2, degi2, W2, b2r, W_out, bor)


def kernel(x, edge_index, W1, b1, W2, b2, W_out, b_out):
    src = edge_index[0]
    dst = edge_index[1]
    pad_ids = jnp.arange(EPAD - E, dtype=jnp.int32) % NPAD_ROWS
    src_p = jnp.concatenate([src, pad_ids])
    dst_p = jnp.concatenate([dst, AGG_OUT_ROWS + pad_ids])
    packed = src_p + dst_p * (1 << 14)

    dego, degi = _hist(src, dst)
    dego2 = dego[:N].reshape(N, 1)
    degi2 = degi[:N].reshape(N, 1)

    xs = _prep(x, dego2)
    agg1 = _agg(xs, packed, D_IN // 128)
    h1s = _mm1(agg1, degi2, dego2, W1, b1.reshape(1, -1))
    agg2 = _agg(h1s, packed, D_H // 128)
    return _mm2(agg2, degi2, W2, b2.reshape(1, -1), W_out, b_out.reshape(1, -1))


# speedup vs baseline: 5.9961x; 1.4772x over previous
import functools

import jax
import jax.numpy as jnp
from jax import lax
from jax.experimental import pallas as pl
from jax.experimental.pallas import tpu as pltpu
from jax.experimental.pallas import tpu_sc as plsc

N = 10000
E = 160000
D_IN = 256
D_H = 1024
D_OUT = 128

NCORES = 2
NTILES = 16
K = 128
EPAD = 163840
CHUNKS_PER_TILE = EPAD // (K * NTILES)
NPAD_ROWS = 48
SHARE_ROWS = 632
AGG_OUT_ROWS = NTILES * SHARE_ROWS
NP1 = AGG_OUT_ROWS + NPAD_ROWS
DEG_PAD = 10240
DEG_SHARE = DEG_PAD // NTILES

BN = 1000
NB_N = N // BN


def _hist_body(srcp, dstp, dego, degi, deg_sp, idx_v, ones_v, zer_v):
    cid = lax.axis_index("c")
    sid = lax.axis_index("s")
    for i in range(8):
        ones_v[pl.ds(i * 16, 16)] = jnp.ones((16,), jnp.float32)

    def zfill(i, c):
        zer_v[pl.ds(i * 16, 16)] = jnp.zeros((16,), jnp.float32)
        return c
    lax.fori_loop(0, DEG_SHARE // 16, zfill, 0)

    base = pl.multiple_of(sid * DEG_SHARE, 8)
    pltpu.sync_copy(zer_v, deg_sp.at[pl.ds(base, DEG_SHARE)])
    plsc.subcore_barrier()

    total_chunks = E // K
    nbase, nrem = divmod(total_chunks, NTILES)
    n_j = nbase + jnp.where(sid < nrem, 1, 0)

    def accumulate(edge_ref):
        def body(j, c):
            e0 = pl.multiple_of((sid + NTILES * j) * K, 8)
            pltpu.sync_copy(edge_ref.at[pl.ds(e0, K)], idx_v)
            pltpu.sync_copy(ones_v, deg_sp.at[idx_v], add=True)
            return c
        lax.fori_loop(0, n_j, body, 0)

    @pl.when(cid == 0)
    def _():
        accumulate(srcp)

    @pl.when(cid == 1)
    def _():
        accumulate(dstp)

    plsc.subcore_barrier()

    @pl.when(cid == 0)
    def _():
        pltpu.sync_copy(deg_sp.at[pl.ds(base, DEG_SHARE)],
                        dego.at[pl.ds(base, DEG_SHARE)])

    @pl.when(cid == 1)
    def _():
        pltpu.sync_copy(deg_sp.at[pl.ds(base, DEG_SHARE)],
                        degi.at[pl.ds(base, DEG_SHARE)])


def _hist(src_p, dst_p):
    mesh = plsc.VectorSubcoreMesh(core_axis_name="c", subcore_axis_name="s")
    return pl.kernel(
        _hist_body,
        out_type=(pltpu.HBM((DEG_PAD,), jnp.float32),
                  pltpu.HBM((DEG_PAD,), jnp.float32)),
        mesh=mesh,
        scratch_types=[
            pltpu.VMEM_SHARED((DEG_PAD,), jnp.float32),
            pltpu.VMEM((K,), jnp.int32),
            pltpu.VMEM((K,), jnp.float32),
            pltpu.VMEM((DEG_SHARE,), jnp.float32),
        ],
    )(src_p, dst_p)


_EDGES_PER_TILE = EPAD // NTILES


def _unpack_chunk(pk_v, j, sidx, didx):
    for t in range(K // 16):
        pk = pk_v[pl.ds(j * K + t * 16, 16)]
        didx[pl.ds(t * 16, 16)] = lax.shift_right_logical(pk, 14)
        sidx[pl.ds(t * 16, 16)] = lax.bitwise_and(pk, (1 << 14) - 1)


def _agg_body(nblk_per_core, xb, packed, zeros_hbm, out,
              agg_sp, pk_v, rows0, rows1, sidx0, didx0, sidx1, didx1,
              g0s, g1s, s0s, s1s):
    cid = lax.axis_index("c")
    sid = lax.axis_index("s")

    ebase = pl.multiple_of(sid * _EDGES_PER_TILE, 8)
    pltpu.sync_copy(packed.at[pl.ds(ebase, _EDGES_PER_TILE)], pk_v)

    rbase = sid * SHARE_ROWS
    for b in range(nblk_per_core):
        bg = cid * nblk_per_core + b
        pltpu.sync_copy(zeros_hbm, agg_sp.at[pl.ds(rbase, SHARE_ROWS)])
        plsc.subcore_barrier()

        xrows = xb.at[bg]

        def pair(p, c):
            _unpack_chunk(pk_v, 2 * p, sidx0, didx0)
            _unpack_chunk(pk_v, 2 * p + 1, sidx1, didx1)
            g0 = pltpu.async_copy(xrows.at[sidx0], rows0, g0s)
            g1 = pltpu.async_copy(xrows.at[sidx1], rows1, g1s)
            g0.wait()
            s0 = pltpu.async_copy(rows0, agg_sp.at[didx0], s0s, add=True)
            g1.wait()
            s1 = pltpu.async_copy(rows1, agg_sp.at[didx1], s1s, add=True)
            s0.wait()
            s1.wait()
            return c
        lax.fori_loop(0, CHUNKS_PER_TILE // 2, pair, 0)
        plsc.subcore_barrier()

        pltpu.sync_copy(agg_sp.at[pl.ds(rbase, SHARE_ROWS)],
                        out.at[pl.ds(rbase, SHARE_ROWS), pl.ds(bg * 128, 128)])
        plsc.subcore_barrier()


def _agg(xb, packed, nblocks):
    mesh = plsc.VectorSubcoreMesh(core_axis_name="c", subcore_axis_name="s")
    return pl.kernel(
        functools.partial(_agg_body, nblocks // NCORES),
        out_type=pltpu.HBM((AGG_OUT_ROWS, nblocks * 128), jnp.float32),
        mesh=mesh,
        scratch_types=[
            pltpu.VMEM_SHARED((NP1, 128), jnp.float32),
            pltpu.VMEM((_EDGES_PER_TILE,), jnp.int32),
            pltpu.VMEM((K, 128), jnp.float32),
            pltpu.VMEM((K, 128), jnp.float32),
            pltpu.VMEM((K,), jnp.int32),
            pltpu.VMEM((K,), jnp.int32),
            pltpu.VMEM((K,), jnp.int32),
            pltpu.VMEM((K,), jnp.int32),
            pltpu.SemaphoreType.DMA,
            pltpu.SemaphoreType.DMA,
            pltpu.SemaphoreType.DMA,
            pltpu.SemaphoreType.DMA,
        ],
    )(xb, packed, jnp.zeros((SHARE_ROWS, 128), jnp.float32))


def _prep_body(x_ref, dego_ref, xs_ref):
    s = lax.rsqrt(jnp.maximum(dego_ref[...], 1.0))
    xs_ref[0] = x_ref[...] * s


def _prep(x, dego2):
    nb_f = D_IN // 128
    return pl.pallas_call(
        _prep_body,
        grid=(nb_f, NB_N),
        in_specs=[
            pl.BlockSpec((BN, 128), lambda b, n: (n, b)),
            pl.BlockSpec((BN, 1), lambda b, n: (n, 0)),
        ],
        out_specs=pl.BlockSpec((1, BN, 128), lambda b, n: (b, n, 0)),
        out_shape=jax.ShapeDtypeStruct((nb_f, N, 128), jnp.float32),
    )(x, dego2)


def _mm1_body(agg_ref, degi_ref, dego_ref, w_ref, b_ref, out_ref):
    acc = jnp.dot(agg_ref[...], w_ref[...], preferred_element_type=jnp.float32)
    s_in = lax.rsqrt(jnp.maximum(degi_ref[...], 1.0))
    s_out = lax.rsqrt(jnp.maximum(dego_ref[...], 1.0))
    out_ref[0] = jnp.maximum(acc * s_in + b_ref[...], 0.0) * s_out


def _mm1(agg1, degi2, dego2, W1, b1r):
    no = D_H // 128
    return pl.pallas_call(
        _mm1_body,
        grid=(NB_N, no),
        in_specs=[
            pl.BlockSpec((BN, D_IN), lambda n, o: (n, 0)),
            pl.BlockSpec((BN, 1), lambda n, o: (n, 0)),
            pl.BlockSpec((BN, 1), lambda n, o: (n, 0)),
            pl.BlockSpec((D_IN, 128), lambda n, o: (0, o)),
            pl.BlockSpec((1, 128), lambda n, o: (0, o)),
        ],
        out_specs=pl.BlockSpec((1, BN, 128), lambda n, o: (o, n, 0)),
        out_shape=jax.ShapeDtypeStruct((no, N, 128), jnp.float32),
    )(agg1, degi2, dego2, W1, b1r)


def _mm2_body(agg_ref, degi_ref, w2_ref, b2_ref, wo_ref, bo_ref, out_ref,
              csum_ref):
    n = pl.program_id(0)
    o = pl.program_id(1)
    no = pl.num_programs(1)

    acc = jnp.dot(agg_ref[...], w2_ref[...], preferred_element_type=jnp.float32)
    s_in = lax.rsqrt(jnp.maximum(degi_ref[...], 1.0))
    h = jnp.maximum(acc * s_in + b2_ref[...], 0.0)
    part = jnp.sum(h, axis=0, keepdims=True)

    for oi in range(no):
        @pl.when(jnp.logical_and(o == oi, n == 0))
        def _():
            csum_ref[pl.ds(oi, 1), :] = part

        @pl.when(jnp.logical_and(o == oi, n > 0))
        def _():
            csum_ref[pl.ds(oi, 1), :] += part

    @pl.when(n == NB_N - 1)
    def _():
        for oi in range(no):
            @pl.when(o == oi)
            def _():
                mean_o = csum_ref[pl.ds(oi, 1), :] * (1.0 / N)
                contrib = jnp.dot(mean_o, wo_ref[...],
                                  preferred_element_type=jnp.float32)

                @pl.when(o == 0)
                def _():
                    out_ref[...] = contrib + bo_ref[...]

                @pl.when(o > 0)
                def _():
                    out_ref[...] += contrib


def _mm2(agg2, degi2, W2, b2r, W_out, bor):
    no = D_H // 128
    return pl.pallas_call(
        _mm2_body,
        grid=(NB_N, no),
        in_specs=[
            pl.BlockSpec((BN, D_H), lambda n, o: (n, 0)),
            pl.BlockSpec((BN, 1), lambda n, o: (n, 0)),
            pl.BlockSpec((D_H, 128), lambda n, o: (0, o)),
            pl.BlockSpec((1, 128), lambda n, o: (0, o)),
            pl.BlockSpec((128, D_OUT), lambda n, o: (o, 0)),
            pl.BlockSpec((1, D_OUT), lambda n, o: (0, 0)),
        ],
        out_specs=pl.BlockSpec((1, D_OUT), lambda n, o: (0, 0)),
        out_shape=jax.ShapeDtypeStruct((1, D_OUT), jnp.float32),
        scratch_shapes=[pltpu.VMEM((no, 128), jnp.float32)],
    )(agg2, degi2, W2, b2r, W_out, bor)


def kernel(x, edge_index, W1, b1, W2, b2, W_out, b_out):
    src = edge_index[0]
    dst = edge_index[1]
    pad_ids = jnp.arange(EPAD - E, dtype=jnp.int32) % NPAD_ROWS
    src_p = jnp.concatenate([src, pad_ids])
    dst_p = jnp.concatenate([dst, AGG_OUT_ROWS + pad_ids])
    packed = src_p + dst_p * (1 << 14)

    dego, degi = _hist(src, dst)
    dego2 = dego[:N].reshape(N, 1)
    degi2 = degi[:N].reshape(N, 1)

    xs = _prep(x, dego2)
    agg1 = _agg(xs, packed, D_IN // 128)
    h1s = _mm1(agg1, degi2, dego2, W1, b1.reshape(1, -1))
    agg2 = _agg(h1s, packed, D_H // 128)
    return _mm2(agg2, degi2, W2, b2.reshape(1, -1), W_out, b_out.reshape(1, -1))


# speedup vs baseline: 6.5679x; 1.0954x over previous
import functools

import jax
import jax.numpy as jnp
from jax import lax
from jax.experimental import pallas as pl
from jax.experimental.pallas import tpu as pltpu
from jax.experimental.pallas import tpu_sc as plsc

N = 10000
E = 160000
D_IN = 256
D_H = 1024
D_OUT = 128

NCORES = 2
NTILES = 16
K = 128
EPAD = 163840
CHUNKS_PER_TILE = EPAD // (K * NTILES)
NPAD_ROWS = 48
SHARE_ROWS = 632
AGG_OUT_ROWS = NTILES * SHARE_ROWS
NP1 = AGG_OUT_ROWS + NPAD_ROWS
DEG_PAD = 10240
DEG_SHARE = DEG_PAD // NTILES

BN = 1000
NB_N = N // BN


def _hist_body(srcp, dstp, dego, degi, deg_sp, idx_v, ones_v, zer_v):
    cid = lax.axis_index("c")
    sid = lax.axis_index("s")
    for i in range(8):
        ones_v[pl.ds(i * 16, 16)] = jnp.ones((16,), jnp.float32)

    def zfill(i, c):
        zer_v[pl.ds(i * 16, 16)] = jnp.zeros((16,), jnp.float32)
        return c
    lax.fori_loop(0, DEG_SHARE // 16, zfill, 0)

    base = pl.multiple_of(sid * DEG_SHARE, 8)
    pltpu.sync_copy(zer_v, deg_sp.at[pl.ds(base, DEG_SHARE)])
    plsc.subcore_barrier()

    total_chunks = E // K
    nbase, nrem = divmod(total_chunks, NTILES)
    n_j = nbase + jnp.where(sid < nrem, 1, 0)

    def accumulate(edge_ref):
        def body(j, c):
            e0 = pl.multiple_of((sid + NTILES * j) * K, 8)
            pltpu.sync_copy(edge_ref.at[pl.ds(e0, K)], idx_v)
            pltpu.sync_copy(ones_v, deg_sp.at[idx_v], add=True)
            return c
        lax.fori_loop(0, n_j, body, 0)

    @pl.when(cid == 0)
    def _():
        accumulate(srcp)

    @pl.when(cid == 1)
    def _():
        accumulate(dstp)

    plsc.subcore_barrier()

    @pl.when(cid == 0)
    def _():
        pltpu.sync_copy(deg_sp.at[pl.ds(base, DEG_SHARE)],
                        dego.at[pl.ds(base, DEG_SHARE)])

    @pl.when(cid == 1)
    def _():
        pltpu.sync_copy(deg_sp.at[pl.ds(base, DEG_SHARE)],
                        degi.at[pl.ds(base, DEG_SHARE)])


def _hist(src_p, dst_p):
    mesh = plsc.VectorSubcoreMesh(core_axis_name="c", subcore_axis_name="s")
    return pl.kernel(
        _hist_body,
        out_type=(pltpu.HBM((DEG_PAD,), jnp.float32),
                  pltpu.HBM((DEG_PAD,), jnp.float32)),
        mesh=mesh,
        scratch_types=[
            pltpu.VMEM_SHARED((DEG_PAD,), jnp.float32),
            pltpu.VMEM((K,), jnp.int32),
            pltpu.VMEM((K,), jnp.float32),
            pltpu.VMEM((DEG_SHARE,), jnp.float32),
        ],
    )(src_p, dst_p)


_EDGES_PER_TILE = EPAD // NTILES


def _unpack_chunk(pk_v, j, sidx, didx):
    for t in range(K // 16):
        pk = pk_v[pl.ds(j * K + t * 16, 16)]
        didx[pl.ds(t * 16, 16)] = lax.shift_right_logical(pk, 14)
        sidx[pl.ds(t * 16, 16)] = lax.bitwise_and(pk, (1 << 14) - 1)


def _agg_body(nblk_per_core, xb, packed, zeros_hbm, out,
              agg_sp, pk_v, rows0, rows1, sidx0, didx0, sidx1, didx1,
              g0s, g1s, s0s, s1s):
    cid = lax.axis_index("c")
    sid = lax.axis_index("s")

    ebase = pl.multiple_of(sid * _EDGES_PER_TILE, 8)
    pltpu.sync_copy(packed.at[pl.ds(ebase, _EDGES_PER_TILE)], pk_v)

    rbase = sid * SHARE_ROWS
    for b in range(nblk_per_core):
        bg = cid * nblk_per_core + b
        pltpu.sync_copy(zeros_hbm, agg_sp.at[pl.ds(rbase, SHARE_ROWS)])
        plsc.subcore_barrier()

        xrows = xb.at[bg]

        def phase(j, reuse, rows, sidx, didx, gs, ss):
            @pl.when(reuse)
            def _():
                pltpu.make_async_copy(rows, agg_sp.at[didx], ss).wait()
            _unpack_chunk(pk_v, j, sidx, didx)
            pltpu.async_copy(xrows.at[sidx], rows, gs).wait()
            pltpu.async_copy(rows, agg_sp.at[didx], ss, add=True)

        def pair(p, c):
            phase(2 * p, p >= 1, rows0, sidx0, didx0, g0s, s0s)
            phase(2 * p + 1, p >= 1, rows1, sidx1, didx1, g1s, s1s)
            return c
        lax.fori_loop(0, CHUNKS_PER_TILE // 2, pair, 0)
        pltpu.make_async_copy(rows0, agg_sp.at[didx0], s0s).wait()
        pltpu.make_async_copy(rows1, agg_sp.at[didx1], s1s).wait()
        plsc.subcore_barrier()

        pltpu.sync_copy(agg_sp.at[pl.ds(rbase, SHARE_ROWS)],
                        out.at[pl.ds(rbase, SHARE_ROWS), pl.ds(bg * 128, 128)])
        plsc.subcore_barrier()


def _agg(xb, packed, nblocks):
    mesh = plsc.VectorSubcoreMesh(core_axis_name="c", subcore_axis_name="s")
    return pl.kernel(
        functools.partial(_agg_body, nblocks // NCORES),
        out_type=pltpu.HBM((AGG_OUT_ROWS, nblocks * 128), jnp.float32),
        mesh=mesh,
        scratch_types=[
            pltpu.VMEM_SHARED((NP1, 128), jnp.float32),
            pltpu.VMEM((_EDGES_PER_TILE,), jnp.int32),
            pltpu.VMEM((K, 128), jnp.float32),
            pltpu.VMEM((K, 128), jnp.float32),
            pltpu.VMEM((K,), jnp.int32),
            pltpu.VMEM((K,), jnp.int32),
            pltpu.VMEM((K,), jnp.int32),
            pltpu.VMEM((K,), jnp.int32),
            pltpu.SemaphoreType.DMA,
            pltpu.SemaphoreType.DMA,
            pltpu.SemaphoreType.DMA,
            pltpu.SemaphoreType.DMA,
        ],
    )(xb, packed, jnp.zeros((SHARE_ROWS, 128), jnp.float32))


def _prep_body(x_ref, dego_ref, xs_ref):
    s = lax.rsqrt(jnp.maximum(dego_ref[...], 1.0))
    xs_ref[0] = x_ref[...] * s


def _prep(x, dego2):
    nb_f = D_IN // 128
    return pl.pallas_call(
        _prep_body,
        grid=(nb_f, NB_N),
        in_specs=[
            pl.BlockSpec((BN, 128), lambda b, n: (n, b)),
            pl.BlockSpec((BN, 1), lambda b, n: (n, 0)),
        ],
        out_specs=pl.BlockSpec((1, BN, 128), lambda b, n: (b, n, 0)),
        out_shape=jax.ShapeDtypeStruct((nb_f, N, 128), jnp.float32),
    )(x, dego2)


def _mm1_body(agg_ref, degi_ref, dego_ref, w_ref, b_ref, out_ref):
    acc = jnp.dot(agg_ref[...], w_ref[...], preferred_element_type=jnp.float32)
    s_in = lax.rsqrt(jnp.maximum(degi_ref[...], 1.0))
    s_out = lax.rsqrt(jnp.maximum(dego_ref[...], 1.0))
    out_ref[0] = jnp.maximum(acc * s_in + b_ref[...], 0.0) * s_out


def _mm1(agg1, degi2, dego2, W1, b1r):
    no = D_H // 128
    return pl.pallas_call(
        _mm1_body,
        grid=(NB_N, no),
        in_specs=[
            pl.BlockSpec((BN, D_IN), lambda n, o: (n, 0)),
            pl.BlockSpec((BN, 1), lambda n, o: (n, 0)),
            pl.BlockSpec((BN, 1), lambda n, o: (n, 0)),
            pl.BlockSpec((D_IN, 128), lambda n, o: (0, o)),
            pl.BlockSpec((1, 128), lambda n, o: (0, o)),
        ],
        out_specs=pl.BlockSpec((1, BN, 128), lambda n, o: (o, n, 0)),
        out_shape=jax.ShapeDtypeStruct((no, N, 128), jnp.float32),
    )(agg1, degi2, dego2, W1, b1r)


def _mm2_body(agg_ref, degi_ref, w2_ref, b2_ref, wo_ref, bo_ref, out_ref,
              csum_ref):
    n = pl.program_id(0)
    o = pl.program_id(1)
    no = pl.num_programs(1)

    acc = jnp.dot(agg_ref[...], w2_ref[...], preferred_element_type=jnp.float32)
    s_in = lax.rsqrt(jnp.maximum(degi_ref[...], 1.0))
    h = jnp.maximum(acc * s_in + b2_ref[...], 0.0)
    part = jnp.sum(h, axis=0, keepdims=True)

    for oi in range(no):
        @pl.when(jnp.logical_and(o == oi, n == 0))
        def _():
            csum_ref[pl.ds(oi, 1), :] = part

        @pl.when(jnp.logical_and(o == oi, n > 0))
        def _():
            csum_ref[pl.ds(oi, 1), :] += part

    @pl.when(n == NB_N - 1)
    def _():
        for oi in range(no):
            @pl.when(o == oi)
            def _():
                mean_o = csum_ref[pl.ds(oi, 1), :] * (1.0 / N)
                contrib = jnp.dot(mean_o, wo_ref[...],
                                  preferred_element_type=jnp.float32)

                @pl.when(o == 0)
                def _():
                    out_ref[...] = contrib + bo_ref[...]

                @pl.when(o > 0)
                def _():
                    out_ref[...] += contrib


def _mm2(agg2, degi2, W2, b2r, W_out, bor):
    no = D_H // 128
    return pl.pallas_call(
        _mm2_body,
        grid=(NB_N, no),
        in_specs=[
            pl.BlockSpec((BN, D_H), lambda n, o: (n, 0)),
            pl.BlockSpec((BN, 1), lambda n, o: (n, 0)),
            pl.BlockSpec((D_H, 128), lambda n, o: (0, o)),
            pl.BlockSpec((1, 128), lambda n, o: (0, o)),
            pl.BlockSpec((128, D_OUT), lambda n, o: (o, 0)),
            pl.BlockSpec((1, D_OUT), lambda n, o: (0, 0)),
        ],
        out_specs=pl.BlockSpec((1, D_OUT), lambda n, o: (0, 0)),
        out_shape=jax.ShapeDtypeStruct((1, D_OUT), jnp.float32),
        scratch_shapes=[pltpu.VMEM((no, 128), jnp.float32)],
    )(agg2, degi2, W2, b2r, W_out, bor)


def kernel(x, edge_index, W1, b1, W2, b2, W_out, b_out):
    src = edge_index[0]
    dst = edge_index[1]
    pad_ids = jnp.arange(EPAD - E, dtype=jnp.int32) % NPAD_ROWS
    src_p = jnp.concatenate([src, pad_ids])
    dst_p = jnp.concatenate([dst, AGG_OUT_ROWS + pad_ids])
    packed = src_p + dst_p * (1 << 14)

    dego, degi = _hist(src, dst)
    dego2 = dego[:N].reshape(N, 1)
    degi2 = degi[:N].reshape(N, 1)

    xs = _prep(x, dego2)
    agg1 = _agg(xs, packed, D_IN // 128)
    h1s = _mm1(agg1, degi2, dego2, W1, b1.reshape(1, -1))
    agg2 = _agg(h1s, packed, D_H // 128)
    return _mm2(agg2, degi2, W2, b2.reshape(1, -1), W_out, b_out.reshape(1, -1))


# speedup vs baseline: 7.9216x; 1.2061x over previous
import functools

import jax
import jax.numpy as jnp
from jax import lax
from jax.experimental import pallas as pl
from jax.experimental.pallas import tpu as pltpu
from jax.experimental.pallas import tpu_sc as plsc

N = 10000
E = 160000
D_IN = 256
D_H = 1024
D_OUT = 128

NCORES = 2
NTILES = 16
K = 80
EPAD = 163840
CHUNKS_PER_TILE = EPAD // (K * NTILES)
NPAD_ROWS = 48
SHARE_ROWS = 632
AGG_OUT_ROWS = NTILES * SHARE_ROWS
NP1 = AGG_OUT_ROWS + NPAD_ROWS
DEG_PAD = 10240
DEG_SHARE = DEG_PAD // NTILES

BN = 1000
NB_N = N // BN


HK = 128


def _hist_body(srcp, dstp, dego, degi, deg_sp, idx_v, ones_v, zer_v):
    cid = lax.axis_index("c")
    sid = lax.axis_index("s")
    for i in range(HK // 16):
        ones_v[pl.ds(i * 16, 16)] = jnp.ones((16,), jnp.float32)

    def zfill(i, c):
        zer_v[pl.ds(i * 16, 16)] = jnp.zeros((16,), jnp.float32)
        return c
    lax.fori_loop(0, DEG_SHARE // 16, zfill, 0)

    base = pl.multiple_of(sid * DEG_SHARE, 8)
    pltpu.sync_copy(zer_v, deg_sp.at[pl.ds(base, DEG_SHARE)])
    plsc.subcore_barrier()

    total_chunks = E // HK
    nbase, nrem = divmod(total_chunks, NTILES)
    n_j = nbase + jnp.where(sid < nrem, 1, 0)

    def accumulate(edge_ref):
        def body(j, c):
            e0 = pl.multiple_of((sid + NTILES * j) * HK, 8)
            pltpu.sync_copy(edge_ref.at[pl.ds(e0, HK)], idx_v)
            pltpu.sync_copy(ones_v, deg_sp.at[idx_v], add=True)
            return c
        lax.fori_loop(0, n_j, body, 0)

    @pl.when(cid == 0)
    def _():
        accumulate(srcp)

    @pl.when(cid == 1)
    def _():
        accumulate(dstp)

    plsc.subcore_barrier()

    @pl.when(cid == 0)
    def _():
        pltpu.sync_copy(deg_sp.at[pl.ds(base, DEG_SHARE)],
                        dego.at[pl.ds(base, DEG_SHARE)])

    @pl.when(cid == 1)
    def _():
        pltpu.sync_copy(deg_sp.at[pl.ds(base, DEG_SHARE)],
                        degi.at[pl.ds(base, DEG_SHARE)])


def _hist(src_p, dst_p):
    mesh = plsc.VectorSubcoreMesh(core_axis_name="c", subcore_axis_name="s")
    return pl.kernel(
        _hist_body,
        out_type=(pltpu.HBM((DEG_PAD,), jnp.float32),
                  pltpu.HBM((DEG_PAD,), jnp.float32)),
        mesh=mesh,
        scratch_types=[
            pltpu.VMEM_SHARED((DEG_PAD,), jnp.float32),
            pltpu.VMEM((HK,), jnp.int32),
            pltpu.VMEM((HK,), jnp.float32),
            pltpu.VMEM((DEG_SHARE,), jnp.float32),
        ],
    )(src_p, dst_p)


_EDGES_PER_TILE = EPAD // NTILES


def _unpack_chunk(pk_v, j, sidx, didx):
    for t in range(K // 16):
        pk = pk_v[pl.ds(j * K + t * 16, 16)]
        didx[pl.ds(t * 16, 16)] = lax.shift_right_logical(pk, 14)
        sidx[pl.ds(t * 16, 16)] = lax.bitwise_and(pk, (1 << 14) - 1)


def _agg_body(nblk_per_core, xb, packed, zeros_hbm, out,
              agg_sp, pk_v, rows, sidx, didx, gs, ss):
    cid = lax.axis_index("c")
    sid = lax.axis_index("s")
    nch = CHUNKS_PER_TILE

    ebase = pl.multiple_of(sid * _EDGES_PER_TILE, 8)
    pltpu.sync_copy(packed.at[pl.ds(ebase, _EDGES_PER_TILE)], pk_v)

    rbase = sid * SHARE_ROWS
    for b in range(nblk_per_core):
        bg = cid * nblk_per_core + b
        pltpu.sync_copy(zeros_hbm, agg_sp.at[pl.ds(rbase, SHARE_ROWS)])
        plsc.subcore_barrier()

        xrows = xb.at[bg]

        def prefetch(j1, q1):
            @pl.when(j1 >= 3)
            def _():
                pltpu.make_async_copy(rows[q1], agg_sp.at[didx[q1]],
                                      ss[q1]).wait()
            _unpack_chunk(pk_v, j1, sidx[q1], didx[q1])
            pltpu.async_copy(xrows.at[sidx[q1]], rows[q1], gs[q1])

        def phase(j, q):
            @pl.when(j < nch)
            def _():
                @pl.when(j + 1 < nch)
                def _():
                    prefetch(j + 1, (q + 1) % 3)
                pltpu.make_async_copy(xrows.at[sidx[q]], rows[q],
                                      gs[q]).wait()
                pltpu.async_copy(rows[q], agg_sp.at[didx[q]], ss[q], add=True)

        prefetch(0, 0)

        def body(p, c):
            j = 3 * p
            phase(j, 0)
            phase(j + 1, 1)
            phase(j + 2, 2)
            return c
        lax.fori_loop(0, (nch + 2) // 3, body, 0)
        for q in range(3):
            pltpu.make_async_copy(rows[q], agg_sp.at[didx[q]], ss[q]).wait()
        plsc.subcore_barrier()

        pltpu.sync_copy(agg_sp.at[pl.ds(rbase, SHARE_ROWS)],
                        out.at[pl.ds(rbase, SHARE_ROWS), pl.ds(bg * 128, 128)])
        plsc.subcore_barrier()


def _agg(xb, packed, nblocks):
    mesh = plsc.VectorSubcoreMesh(core_axis_name="c", subcore_axis_name="s")

    def body(xb_r, packed_r, zeros_r, out_r, agg_sp, pk_v,
             r0, r1, r2, si0, si1, si2, di0, di1, di2,
             g0, g1, g2, s0, s1, s2):
        _agg_body(nblocks // NCORES, xb_r, packed_r, zeros_r, out_r,
                  agg_sp, pk_v, (r0, r1, r2), (si0, si1, si2),
                  (di0, di1, di2), (g0, g1, g2), (s0, s1, s2))

    return pl.kernel(
        body,
        out_type=pltpu.HBM((AGG_OUT_ROWS, nblocks * 128), jnp.float32),
        mesh=mesh,
        scratch_types=[
            pltpu.VMEM_SHARED((NP1, 128), jnp.float32),
            pltpu.VMEM((_EDGES_PER_TILE,), jnp.int32),
            pltpu.VMEM((K, 128), jnp.float32),
            pltpu.VMEM((K, 128), jnp.float32),
            pltpu.VMEM((K, 128), jnp.float32),
            pltpu.VMEM((K,), jnp.int32),
            pltpu.VMEM((K,), jnp.int32),
            pltpu.VMEM((K,), jnp.int32),
            pltpu.VMEM((K,), jnp.int32),
            pltpu.VMEM((K,), jnp.int32),
            pltpu.VMEM((K,), jnp.int32),
            pltpu.SemaphoreType.DMA,
            pltpu.SemaphoreType.DMA,
            pltpu.SemaphoreType.DMA,
            pltpu.SemaphoreType.DMA,
            pltpu.SemaphoreType.DMA,
            pltpu.SemaphoreType.DMA,
        ],
    )(xb, packed, jnp.zeros((SHARE_ROWS, 128), jnp.float32))


def _prep_body(x_ref, dego_ref, xs_ref):
    s = lax.rsqrt(jnp.maximum(dego_ref[...], 1.0))
    xs_ref[0] = x_ref[...] * s


def _prep(x, dego2):
    nb_f = D_IN // 128
    return pl.pallas_call(
        _prep_body,
        grid=(nb_f, NB_N),
        in_specs=[
            pl.BlockSpec((BN, 128), lambda b, n: (n, b)),
            pl.BlockSpec((BN, 1), lambda b, n: (n, 0)),
        ],
        out_specs=pl.BlockSpec((1, BN, 128), lambda b, n: (b, n, 0)),
        out_shape=jax.ShapeDtypeStruct((nb_f, N, 128), jnp.float32),
    )(x, dego2)


def _mm1_body(agg_ref, degi_ref, dego_ref, w_ref, b_ref, out_ref):
    acc = jnp.dot(agg_ref[...], w_ref[...], preferred_element_type=jnp.float32)
    s_in = lax.rsqrt(jnp.maximum(degi_ref[...], 1.0))
    s_out = lax.rsqrt(jnp.maximum(dego_ref[...], 1.0))
    out_ref[0] = jnp.maximum(acc * s_in + b_ref[...], 0.0) * s_out


def _mm1(agg1, degi2, dego2, W1, b1r):
    no = D_H // 128
    return pl.pallas_call(
        _mm1_body,
        grid=(NB_N, no),
        in_specs=[
            pl.BlockSpec((BN, D_IN), lambda n, o: (n, 0)),
            pl.BlockSpec((BN, 1), lambda n, o: (n, 0)),
            pl.BlockSpec((BN, 1), lambda n, o: (n, 0)),
            pl.BlockSpec((D_IN, 128), lambda n, o: (0, o)),
            pl.BlockSpec((1, 128), lambda n, o: (0, o)),
        ],
        out_specs=pl.BlockSpec((1, BN, 128), lambda n, o: (o, n, 0)),
        out_shape=jax.ShapeDtypeStruct((no, N, 128), jnp.float32),
    )(agg1, degi2, dego2, W1, b1r)


def _mm2_body(agg_ref, degi_ref, w2_ref, b2_ref, wo_ref, bo_ref, out_ref,
              csum_ref):
    n = pl.program_id(0)
    o = pl.program_id(1)
    no = pl.num_programs(1)

    acc = jnp.dot(agg_ref[...], w2_ref[...], preferred_element_type=jnp.float32)
    s_in = lax.rsqrt(jnp.maximum(degi_ref[...], 1.0))
    h = jnp.maximum(acc * s_in + b2_ref[...], 0.0)
    part = jnp.sum(h, axis=0, keepdims=True)

    for oi in range(no):
        @pl.when(jnp.logical_and(o == oi, n == 0))
        def _():
            csum_ref[pl.ds(oi, 1), :] = part

        @pl.when(jnp.logical_and(o == oi, n > 0))
        def _():
            csum_ref[pl.ds(oi, 1), :] += part

    @pl.when(n == NB_N - 1)
    def _():
        for oi in range(no):
            @pl.when(o == oi)
            def _():
                mean_o = csum_ref[pl.ds(oi, 1), :] * (1.0 / N)
                contrib = jnp.dot(mean_o, wo_ref[...],
                                  preferred_element_type=jnp.float32)

                @pl.when(o == 0)
                def _():
                    out_ref[...] = contrib + bo_ref[...]

                @pl.when(o > 0)
                def _():
                    out_ref[...] += contrib


def _mm2(agg2, degi2, W2, b2r, W_out, bor):
    no = D_H // 128
    return pl.pallas_call(
        _mm2_body,
        grid=(NB_N, no),
        in_specs=[
            pl.BlockSpec((BN, D_H), lambda n, o: (n, 0)),
            pl.BlockSpec((BN, 1), lambda n, o: (n, 0)),
            pl.BlockSpec((D_H, 128), lambda n, o: (0, o)),
            pl.BlockSpec((1, 128), lambda n, o: (0, o)),
            pl.BlockSpec((128, D_OUT), lambda n, o: (o, 0)),
            pl.BlockSpec((1, D_OUT), lambda n, o: (0, 0)),
        ],
        out_specs=pl.BlockSpec((1, D_OUT), lambda n, o: (0, 0)),
        out_shape=jax.ShapeDtypeStruct((1, D_OUT), jnp.float32),
        scratch_shapes=[pltpu.VMEM((no, 128), jnp.float32)],
    )(agg2, degi2, W2, b2r, W_out, bor)


def kernel(x, edge_index, W1, b1, W2, b2, W_out, b_out):
    src = edge_index[0]
    dst = edge_index[1]
    pad_ids = jnp.arange(EPAD - E, dtype=jnp.int32) % NPAD_ROWS
    src_p = jnp.concatenate([src, pad_ids])
    dst_p = jnp.concatenate([dst, AGG_OUT_ROWS + pad_ids])
    packed = src_p + dst_p * (1 << 14)

    dego, degi = _hist(src, dst)
    dego2 = dego[:N].reshape(N, 1)
    degi2 = degi[:N].reshape(N, 1)

    xs = _prep(x, dego2)
    agg1 = _agg(xs, packed, D_IN // 128)
    h1s = _mm1(agg1, degi2, dego2, W1, b1.reshape(1, -1))
    agg2 = _agg(h1s, packed, D_H // 128)
    return _mm2(agg2, degi2, W2, b2.reshape(1, -1), W_out, b_out.reshape(1, -1))


# speedup vs baseline: 7.9458x; 1.0030x over previous
import functools

import jax
import jax.numpy as jnp
from jax import lax
from jax.experimental import pallas as pl
from jax.experimental.pallas import tpu as pltpu
from jax.experimental.pallas import tpu_sc as plsc

N = 10000
E = 160000
D_IN = 256
D_H = 1024
D_OUT = 128

NCORES = 2
NTILES = 16
K = 80
EPAD = 163840
CHUNKS_PER_TILE = EPAD // (K * NTILES)
NPAD_ROWS = 48
SHARE_ROWS = 632
AGG_OUT_ROWS = NTILES * SHARE_ROWS
NP1 = AGG_OUT_ROWS + NPAD_ROWS
DEG_PAD = 10240
DEG_SHARE = DEG_PAD // NTILES

BN = 1000
NB_N = N // BN


HK = 128


def _hist_body(srcp, dstp, dego, degi, deg_sp, idx_v, ones_v, zer_v):
    cid = lax.axis_index("c")
    sid = lax.axis_index("s")
    for i in range(HK // 16):
        ones_v[pl.ds(i * 16, 16)] = jnp.ones((16,), jnp.float32)

    def zfill(i, c):
        zer_v[pl.ds(i * 16, 16)] = jnp.zeros((16,), jnp.float32)
        return c
    lax.fori_loop(0, DEG_SHARE // 16, zfill, 0)

    base = pl.multiple_of(sid * DEG_SHARE, 8)
    pltpu.sync_copy(zer_v, deg_sp.at[pl.ds(base, DEG_SHARE)])
    plsc.subcore_barrier()

    total_chunks = E // HK
    nbase, nrem = divmod(total_chunks, NTILES)
    n_j = nbase + jnp.where(sid < nrem, 1, 0)

    def accumulate(edge_ref):
        def body(j, c):
            e0 = pl.multiple_of((sid + NTILES * j) * HK, 8)
            pltpu.sync_copy(edge_ref.at[pl.ds(e0, HK)], idx_v)
            pltpu.sync_copy(ones_v, deg_sp.at[idx_v], add=True)
            return c
        lax.fori_loop(0, n_j, body, 0)

    @pl.when(cid == 0)
    def _():
        accumulate(srcp)

    @pl.when(cid == 1)
    def _():
        accumulate(dstp)

    plsc.subcore_barrier()

    @pl.when(cid == 0)
    def _():
        pltpu.sync_copy(deg_sp.at[pl.ds(base, DEG_SHARE)],
                        dego.at[pl.ds(base, DEG_SHARE)])

    @pl.when(cid == 1)
    def _():
        pltpu.sync_copy(deg_sp.at[pl.ds(base, DEG_SHARE)],
                        degi.at[pl.ds(base, DEG_SHARE)])


def _hist(src_p, dst_p):
    mesh = plsc.VectorSubcoreMesh(core_axis_name="c", subcore_axis_name="s")
    return pl.kernel(
        _hist_body,
        out_type=(pltpu.HBM((DEG_PAD,), jnp.float32),
                  pltpu.HBM((DEG_PAD,), jnp.float32)),
        mesh=mesh,
        scratch_types=[
            pltpu.VMEM_SHARED((DEG_PAD,), jnp.float32),
            pltpu.VMEM((HK,), jnp.int32),
            pltpu.VMEM((HK,), jnp.float32),
            pltpu.VMEM((DEG_SHARE,), jnp.float32),
        ],
    )(src_p, dst_p)


_EDGES_PER_TILE = EPAD // NTILES


def _unpack_chunk(pk_v, j, sidx, didx):
    for t in range(K // 16):
        pk = pk_v[pl.ds(j * K + t * 16, 16)]
        didx[pl.ds(t * 16, 16)] = lax.shift_right_logical(pk, 14)
        sidx[pl.ds(t * 16, 16)] = lax.bitwise_and(pk, (1 << 14) - 1)


def _agg_body(nblk_per_core, xb, packed, zeros_hbm, out,
              agg_sp, pk_v, rows, sidx, didx, gs, ss):
    cid = lax.axis_index("c")
    sid = lax.axis_index("s")
    nch = CHUNKS_PER_TILE

    ebase = pl.multiple_of(sid * _EDGES_PER_TILE, 8)
    pltpu.sync_copy(packed.at[pl.ds(ebase, _EDGES_PER_TILE)], pk_v)

    rbase = sid * SHARE_ROWS
    for b in range(nblk_per_core):
        bg = cid * nblk_per_core + b
        pltpu.sync_copy(zeros_hbm, agg_sp.at[pl.ds(rbase, SHARE_ROWS)])
        plsc.subcore_barrier()

        xrows = xb.at[bg]

        def prefetch(j1, q1):
            @pl.when(j1 >= 3)
            def _():
                pltpu.make_async_copy(rows[q1], agg_sp.at[didx[q1]],
                                      ss[q1]).wait()
            _unpack_chunk(pk_v, j1, sidx[q1], didx[q1])
            pltpu.async_copy(xrows.at[sidx[q1]], rows[q1], gs[q1])

        def phase(j, q):
            @pl.when(j < nch)
            def _():
                @pl.when(j + 1 < nch)
                def _():
                    prefetch(j + 1, (q + 1) % 3)
                pltpu.make_async_copy(xrows.at[sidx[q]], rows[q],
                                      gs[q]).wait()
                pltpu.async_copy(rows[q], agg_sp.at[didx[q]], ss[q], add=True)

        prefetch(0, 0)

        def body(p, c):
            j = 3 * p
            phase(j, 0)
            phase(j + 1, 1)
            phase(j + 2, 2)
            return c
        lax.fori_loop(0, (nch + 2) // 3, body, 0)
        for q in range(3):
            pltpu.make_async_copy(rows[q], agg_sp.at[didx[q]], ss[q]).wait()
        plsc.subcore_barrier()

        pltpu.sync_copy(agg_sp.at[pl.ds(rbase, SHARE_ROWS)],
                        out.at[pl.ds(rbase, SHARE_ROWS), pl.ds(bg * 128, 128)])
        plsc.subcore_barrier()


def _agg(xb, packed, nblocks):
    mesh = plsc.VectorSubcoreMesh(core_axis_name="c", subcore_axis_name="s")

    def body(xb_r, packed_r, zeros_r, out_r, agg_sp, pk_v,
             r0, r1, r2, si0, si1, si2, di0, di1, di2,
             g0, g1, g2, s0, s1, s2):
        _agg_body(nblocks // NCORES, xb_r, packed_r, zeros_r, out_r,
                  agg_sp, pk_v, (r0, r1, r2), (si0, si1, si2),
                  (di0, di1, di2), (g0, g1, g2), (s0, s1, s2))

    return pl.kernel(
        body,
        out_type=pltpu.HBM((AGG_OUT_ROWS, nblocks * 128), jnp.float32),
        mesh=mesh,
        scratch_types=[
            pltpu.VMEM_SHARED((NP1, 128), jnp.float32),
            pltpu.VMEM((_EDGES_PER_TILE,), jnp.int32),
            pltpu.VMEM((K, 128), jnp.float32),
            pltpu.VMEM((K, 128), jnp.float32),
            pltpu.VMEM((K, 128), jnp.float32),
            pltpu.VMEM((K,), jnp.int32),
            pltpu.VMEM((K,), jnp.int32),
            pltpu.VMEM((K,), jnp.int32),
            pltpu.VMEM((K,), jnp.int32),
            pltpu.VMEM((K,), jnp.int32),
            pltpu.VMEM((K,), jnp.int32),
            pltpu.SemaphoreType.DMA,
            pltpu.SemaphoreType.DMA,
            pltpu.SemaphoreType.DMA,
            pltpu.SemaphoreType.DMA,
            pltpu.SemaphoreType.DMA,
            pltpu.SemaphoreType.DMA,
        ],
    )(xb, packed, jnp.zeros((SHARE_ROWS, 128), jnp.float32))


def _prep_body(x_ref, dego_ref, xs_ref):
    s = lax.rsqrt(jnp.maximum(dego_ref[...], 1.0))
    xs_ref[0] = x_ref[...] * s


def _prep(x, dego2):
    nb_f = D_IN // 128
    return pl.pallas_call(
        _prep_body,
        grid=(nb_f, NB_N),
        in_specs=[
            pl.BlockSpec((BN, 128), lambda b, n: (n, b)),
            pl.BlockSpec((BN, 1), lambda b, n: (n, 0)),
        ],
        out_specs=pl.BlockSpec((1, BN, 128), lambda b, n: (b, n, 0)),
        out_shape=jax.ShapeDtypeStruct((nb_f, N, 128), jnp.float32),
    )(x, dego2)


def _mm1_body(agg_ref, degi_ref, dego_ref, w_ref, b_ref, out_ref, lhs_ref):
    o = pl.program_id(1)

    @pl.when(o == 0)
    def _():
        lhs_ref[...] = agg_ref[...].astype(jnp.bfloat16)

    acc = jnp.dot(lhs_ref[...], w_ref[...], preferred_element_type=jnp.float32)
    s_in = lax.rsqrt(jnp.maximum(degi_ref[...], 1.0))
    s_out = lax.rsqrt(jnp.maximum(dego_ref[...], 1.0))
    out_ref[0] = jnp.maximum(acc * s_in + b_ref[...], 0.0) * s_out


def _mm1(agg1, degi2, dego2, W1, b1r):
    no = D_H // 128
    return pl.pallas_call(
        _mm1_body,
        grid=(NB_N, no),
        in_specs=[
            pl.BlockSpec((BN, D_IN), lambda n, o: (n, 0)),
            pl.BlockSpec((BN, 1), lambda n, o: (n, 0)),
            pl.BlockSpec((BN, 1), lambda n, o: (n, 0)),
            pl.BlockSpec((D_IN, 128), lambda n, o: (0, o)),
            pl.BlockSpec((1, 128), lambda n, o: (0, o)),
        ],
        out_specs=pl.BlockSpec((1, BN, 128), lambda n, o: (o, n, 0)),
        out_shape=jax.ShapeDtypeStruct((no, N, 128), jnp.float32),
        scratch_shapes=[pltpu.VMEM((BN, D_IN), jnp.bfloat16)],
    )(agg1, degi2, dego2, W1.astype(jnp.bfloat16), b1r)


def _mm2_body(agg_ref, degi_ref, w2_ref, b2_ref, wo_ref, bo_ref, out_ref,
              csum_ref, lhs_ref):
    n = pl.program_id(0)
    o = pl.program_id(1)
    no = pl.num_programs(1)

    @pl.when(o == 0)
    def _():
        lhs_ref[...] = agg_ref[...].astype(jnp.bfloat16)

    acc = jnp.dot(lhs_ref[...], w2_ref[...], preferred_element_type=jnp.float32)
    s_in = lax.rsqrt(jnp.maximum(degi_ref[...], 1.0))
    h = jnp.maximum(acc * s_in + b2_ref[...], 0.0)
    part = jnp.sum(h, axis=0, keepdims=True)

    for oi in range(no):
        @pl.when(jnp.logical_and(o == oi, n == 0))
        def _():
            csum_ref[pl.ds(oi, 1), :] = part

        @pl.when(jnp.logical_and(o == oi, n > 0))
        def _():
            csum_ref[pl.ds(oi, 1), :] += part

    @pl.when(n == NB_N - 1)
    def _():
        for oi in range(no):
            @pl.when(o == oi)
            def _():
                mean_o = csum_ref[pl.ds(oi, 1), :] * (1.0 / N)
                contrib = jnp.dot(mean_o, wo_ref[...],
                                  preferred_element_type=jnp.float32)

                @pl.when(o == 0)
                def _():
                    out_ref[...] = contrib + bo_ref[...]

                @pl.when(o > 0)
                def _():
                    out_ref[...] += contrib


def _mm2(agg2, degi2, W2, b2r, W_out, bor):
    no = D_H // 128
    return pl.pallas_call(
        _mm2_body,
        grid=(NB_N, no),
        in_specs=[
            pl.BlockSpec((BN, D_H), lambda n, o: (n, 0)),
            pl.BlockSpec((BN, 1), lambda n, o: (n, 0)),
            pl.BlockSpec((D_H, 128), lambda n, o: (0, o)),
            pl.BlockSpec((1, 128), lambda n, o: (0, o)),
            pl.BlockSpec((128, D_OUT), lambda n, o: (o, 0)),
            pl.BlockSpec((1, D_OUT), lambda n, o: (0, 0)),
        ],
        out_specs=pl.BlockSpec((1, D_OUT), lambda n, o: (0, 0)),
        out_shape=jax.ShapeDtypeStruct((1, D_OUT), jnp.float32),
        scratch_shapes=[pltpu.VMEM((no, 128), jnp.float32),
                        pltpu.VMEM((BN, D_H), jnp.bfloat16)],
    )(agg2, degi2, W2.astype(jnp.bfloat16), b2r, W_out, bor)


def kernel(x, edge_index, W1, b1, W2, b2, W_out, b_out):
    src = edge_index[0]
    dst = edge_index[1]
    pad_ids = jnp.arange(EPAD - E, dtype=jnp.int32) % NPAD_ROWS
    src_p = jnp.concatenate([src, pad_ids])
    dst_p = jnp.concatenate([dst, AGG_OUT_ROWS + pad_ids])
    packed = src_p + dst_p * (1 << 14)

    dego, degi = _hist(src, dst)
    dego2 = dego[:N].reshape(N, 1)
    degi2 = degi[:N].reshape(N, 1)

    xs = _prep(x, dego2)
    agg1 = _agg(xs, packed, D_IN // 128)
    h1s = _mm1(agg1, degi2, dego2, W1, b1.reshape(1, -1))
    agg2 = _agg(h1s, packed, D_H // 128)
    return _mm2(agg2, degi2, W2, b2.reshape(1, -1), W_out, b_out.reshape(1, -1))


# speedup vs baseline: 8.4902x; 1.0685x over previous
import functools

import jax
import jax.numpy as jnp
from jax import lax
from jax.experimental import pallas as pl
from jax.experimental.pallas import tpu as pltpu
from jax.experimental.pallas import tpu_sc as plsc

N = 10000
E = 160000
D_IN = 256
D_H = 1024
D_OUT = 128

NCORES = 2
NTILES = 16
K = 80
EPAD = 163840
CHUNKS_PER_TILE = EPAD // (K * NTILES)
NPAD_ROWS = 48
SHARE_ROWS = 640
AGG_OUT_ROWS = NTILES * SHARE_ROWS
NP1 = AGG_OUT_ROWS + NPAD_ROWS
DEG_PAD = 10240
DEG_SHARE = DEG_PAD // NTILES

BN = 2000
NB_N = N // BN


HK = 128


def _hist_body(srcp, dstp, dego, degi, deg_sp, idx_v, ones_v, zer_v):
    cid = lax.axis_index("c")
    sid = lax.axis_index("s")
    for i in range(HK // 16):
        ones_v[pl.ds(i * 16, 16)] = jnp.ones((16,), jnp.float32)

    def zfill(i, c):
        zer_v[pl.ds(i * 16, 16)] = jnp.zeros((16,), jnp.float32)
        return c
    lax.fori_loop(0, DEG_SHARE // 16, zfill, 0)

    base = pl.multiple_of(sid * DEG_SHARE, 8)
    pltpu.sync_copy(zer_v, deg_sp.at[pl.ds(base, DEG_SHARE)])
    plsc.subcore_barrier()

    total_chunks = E // HK
    nbase, nrem = divmod(total_chunks, NTILES)
    n_j = nbase + jnp.where(sid < nrem, 1, 0)

    def accumulate(edge_ref):
        def body(j, c):
            e0 = pl.multiple_of((sid + NTILES * j) * HK, 8)
            pltpu.sync_copy(edge_ref.at[pl.ds(e0, HK)], idx_v)
            pltpu.sync_copy(ones_v, deg_sp.at[idx_v], add=True)
            return c
        lax.fori_loop(0, n_j, body, 0)

    @pl.when(cid == 0)
    def _():
        accumulate(srcp)

    @pl.when(cid == 1)
    def _():
        accumulate(dstp)

    plsc.subcore_barrier()

    @pl.when(cid == 0)
    def _():
        pltpu.sync_copy(deg_sp.at[pl.ds(base, DEG_SHARE)],
                        dego.at[pl.ds(base, DEG_SHARE)])

    @pl.when(cid == 1)
    def _():
        pltpu.sync_copy(deg_sp.at[pl.ds(base, DEG_SHARE)],
                        degi.at[pl.ds(base, DEG_SHARE)])


def _hist(src_p, dst_p):
    mesh = plsc.VectorSubcoreMesh(core_axis_name="c", subcore_axis_name="s")
    return pl.kernel(
        _hist_body,
        out_type=(pltpu.HBM((DEG_PAD,), jnp.float32),
                  pltpu.HBM((DEG_PAD,), jnp.float32)),
        mesh=mesh,
        scratch_types=[
            pltpu.VMEM_SHARED((DEG_PAD,), jnp.float32),
            pltpu.VMEM((HK,), jnp.int32),
            pltpu.VMEM((HK,), jnp.float32),
            pltpu.VMEM((DEG_SHARE,), jnp.float32),
        ],
    )(src_p, dst_p)


_EDGES_PER_TILE = EPAD // NTILES


def _unpack_chunk(pk_v, j, sidx, didx):
    for t in range(K // 16):
        pk = pk_v[pl.ds(j * K + t * 16, 16)]
        didx[pl.ds(t * 16, 16)] = lax.shift_right_logical(pk, 14)
        sidx[pl.ds(t * 16, 16)] = lax.bitwise_and(pk, (1 << 14) - 1)


def _agg_body(nblk_per_core, xb, packed, zeros_hbm, out,
              agg_sp, pk_v, rows, sidx, didx, gs, ss):
    cid = lax.axis_index("c")
    sid = lax.axis_index("s")
    nch = CHUNKS_PER_TILE

    ebase = pl.multiple_of(sid * _EDGES_PER_TILE, 8)
    pltpu.sync_copy(packed.at[pl.ds(ebase, _EDGES_PER_TILE)], pk_v)

    rbase = sid * SHARE_ROWS
    for b in range(nblk_per_core):
        bg = cid * nblk_per_core + b
        pltpu.sync_copy(zeros_hbm, agg_sp.at[pl.ds(rbase, SHARE_ROWS)])
        plsc.subcore_barrier()

        xrows = xb.at[bg]

        def prefetch(j1, q1):
            @pl.when(j1 >= 3)
            def _():
                pltpu.make_async_copy(rows[q1], agg_sp.at[didx[q1]],
                                      ss[q1]).wait()
            _unpack_chunk(pk_v, j1, sidx[q1], didx[q1])
            pltpu.async_copy(xrows.at[sidx[q1]], rows[q1], gs[q1])

        def phase(j, q):
            @pl.when(j < nch)
            def _():
                @pl.when(j + 1 < nch)
                def _():
                    prefetch(j + 1, (q + 1) % 3)
                pltpu.make_async_copy(xrows.at[sidx[q]], rows[q],
                                      gs[q]).wait()
                pltpu.async_copy(rows[q], agg_sp.at[didx[q]], ss[q], add=True)

        prefetch(0, 0)

        def body(p, c):
            j = 3 * p
            phase(j, 0)
            phase(j + 1, 1)
            phase(j + 2, 2)
            return c
        lax.fori_loop(0, (nch + 2) // 3, body, 0)
        for q in range(3):
            pltpu.make_async_copy(rows[q], agg_sp.at[didx[q]], ss[q]).wait()
        plsc.subcore_barrier()

        pltpu.sync_copy(agg_sp.at[pl.ds(rbase, SHARE_ROWS)],
                        out.at[pl.ds(rbase, SHARE_ROWS), pl.ds(bg * 128, 128)])
        plsc.subcore_barrier()


def _agg(xb, packed, nblocks, dtype=jnp.float32):
    mesh = plsc.VectorSubcoreMesh(core_axis_name="c", subcore_axis_name="s")

    def body(xb_r, packed_r, zeros_r, out_r, agg_sp, pk_v,
             r0, r1, r2, si0, si1, si2, di0, di1, di2,
             g0, g1, g2, s0, s1, s2):
        _agg_body(nblocks // NCORES, xb_r, packed_r, zeros_r, out_r,
                  agg_sp, pk_v, (r0, r1, r2), (si0, si1, si2),
                  (di0, di1, di2), (g0, g1, g2), (s0, s1, s2))

    return pl.kernel(
        body,
        out_type=pltpu.HBM((AGG_OUT_ROWS, nblocks * 128), dtype),
        mesh=mesh,
        scratch_types=[
            pltpu.VMEM_SHARED((NP1, 128), dtype),
            pltpu.VMEM((_EDGES_PER_TILE,), jnp.int32),
            pltpu.VMEM((K, 128), dtype),
            pltpu.VMEM((K, 128), dtype),
            pltpu.VMEM((K, 128), dtype),
            pltpu.VMEM((K,), jnp.int32),
            pltpu.VMEM((K,), jnp.int32),
            pltpu.VMEM((K,), jnp.int32),
            pltpu.VMEM((K,), jnp.int32),
            pltpu.VMEM((K,), jnp.int32),
            pltpu.VMEM((K,), jnp.int32),
            pltpu.SemaphoreType.DMA,
            pltpu.SemaphoreType.DMA,
            pltpu.SemaphoreType.DMA,
            pltpu.SemaphoreType.DMA,
            pltpu.SemaphoreType.DMA,
            pltpu.SemaphoreType.DMA,
        ],
    )(xb, packed, jnp.zeros((SHARE_ROWS, 128), dtype))


def _prep_body(x_ref, dego_ref, xs_ref):
    s = lax.rsqrt(jnp.maximum(dego_ref[...], 1.0))
    xs_ref[0] = x_ref[...] * s


def _prep(x, dego2):
    nb_f = D_IN // 128
    return pl.pallas_call(
        _prep_body,
        grid=(nb_f, NB_N),
        in_specs=[
            pl.BlockSpec((BN, 128), lambda b, n: (n, b)),
            pl.BlockSpec((BN, 1), lambda b, n: (n, 0)),
        ],
        out_specs=pl.BlockSpec((1, BN, 128), lambda b, n: (b, n, 0)),
        out_shape=jax.ShapeDtypeStruct((nb_f, N, 128), jnp.float32),
    )(x, dego2)


def _mm1_body(agg_ref, degi_ref, dego_ref, w_ref, b_ref, out_ref, lhs_ref):
    o = pl.program_id(1)

    @pl.when(o == 0)
    def _():
        lhs_ref[...] = agg_ref[...].astype(jnp.bfloat16)

    acc = jnp.dot(lhs_ref[...], w_ref[...], preferred_element_type=jnp.float32)
    s_in = lax.rsqrt(jnp.maximum(degi_ref[...], 1.0))
    s_out = lax.rsqrt(jnp.maximum(dego_ref[...], 1.0))
    out_ref[0] = jnp.maximum(acc * s_in + b_ref[...], 0.0) * s_out


def _mm1(agg1, degi2, dego2, W1, b1r):
    no = D_H // 128
    return pl.pallas_call(
        _mm1_body,
        grid=(NB_N, no),
        in_specs=[
            pl.BlockSpec((BN, D_IN), lambda n, o: (n, 0)),
            pl.BlockSpec((BN, 1), lambda n, o: (n, 0)),
            pl.BlockSpec((BN, 1), lambda n, o: (n, 0)),
            pl.BlockSpec((D_IN, 128), lambda n, o: (0, o)),
            pl.BlockSpec((1, 128), lambda n, o: (0, o)),
        ],
        out_specs=pl.BlockSpec((1, BN, 128), lambda n, o: (o, n, 0)),
        out_shape=jax.ShapeDtypeStruct((no, N, 128), jnp.float32),
        scratch_shapes=[pltpu.VMEM((BN, D_IN), jnp.bfloat16)],
    )(agg1, degi2, dego2, W1.astype(jnp.bfloat16), b1r)


def _mm2_body(agg_ref, degi_ref, w2_ref, b2_ref, wo_ref, bo_ref, out_ref,
              csum_ref, lhs_ref):
    n = pl.program_id(0)
    o = pl.program_id(1)
    no = pl.num_programs(1)

    @pl.when(o == 0)
    def _():
        lhs_ref[...] = agg_ref[...].astype(jnp.bfloat16)

    acc = jnp.dot(lhs_ref[...], w2_ref[...], preferred_element_type=jnp.float32)
    s_in = lax.rsqrt(jnp.maximum(degi_ref[...], 1.0))
    h = jnp.maximum(acc * s_in + b2_ref[...], 0.0)
    part = jnp.sum(h, axis=0, keepdims=True)

    for oi in range(no):
        @pl.when(jnp.logical_and(o == oi, n == 0))
        def _():
            csum_ref[pl.ds(oi, 1), :] = part

        @pl.when(jnp.logical_and(o == oi, n > 0))
        def _():
            csum_ref[pl.ds(oi, 1), :] += part

    @pl.when(n == NB_N - 1)
    def _():
        for oi in range(no):
            @pl.when(o == oi)
            def _():
                mean_o = csum_ref[pl.ds(oi, 1), :] * (1.0 / N)
                contrib = jnp.dot(mean_o, wo_ref[...],
                                  preferred_element_type=jnp.float32)

                @pl.when(o == 0)
                def _():
                    out_ref[...] = contrib + bo_ref[...]

                @pl.when(o > 0)
                def _():
                    out_ref[...] += contrib


def _mm2(agg2, degi2, W2, b2r, W_out, bor):
    no = D_H // 128
    return pl.pallas_call(
        _mm2_body,
        grid=(NB_N, no),
        in_specs=[
            pl.BlockSpec((BN, D_H), lambda n, o: (n, 0)),
            pl.BlockSpec((BN, 1), lambda n, o: (n, 0)),
            pl.BlockSpec((D_H, 128), lambda n, o: (0, o)),
            pl.BlockSpec((1, 128), lambda n, o: (0, o)),
            pl.BlockSpec((128, D_OUT), lambda n, o: (o, 0)),
            pl.BlockSpec((1, D_OUT), lambda n, o: (0, 0)),
        ],
        out_specs=pl.BlockSpec((1, D_OUT), lambda n, o: (0, 0)),
        out_shape=jax.ShapeDtypeStruct((1, D_OUT), jnp.float32),
        scratch_shapes=[pltpu.VMEM((no, 128), jnp.float32),
                        pltpu.VMEM((BN, D_H), jnp.bfloat16)],
    )(agg2, degi2, W2.astype(jnp.bfloat16), b2r, W_out, bor)


def kernel(x, edge_index, W1, b1, W2, b2, W_out, b_out):
    src = edge_index[0]
    dst = edge_index[1]
    pad_ids = jnp.arange(EPAD - E, dtype=jnp.int32) % NPAD_ROWS
    src_p = jnp.concatenate([src, pad_ids])
    dst_p = jnp.concatenate([dst, AGG_OUT_ROWS + pad_ids])
    packed = src_p + dst_p * (1 << 14)

    dego, degi = _hist(src, dst)
    dego2 = dego[:N].reshape(N, 1)
    degi2 = degi[:N].reshape(N, 1)

    xs = _prep(x, dego2)
    agg1 = _agg(xs, packed, D_IN // 128)
    h1s = _mm1(agg1, degi2, dego2, W1, b1.reshape(1, -1))
    agg2 = _agg(h1s, packed, D_H // 128)
    return _mm2(agg2, degi2, W2, b2.reshape(1, -1), W_out, b_out.reshape(1, -1))


# speedup vs baseline: 8.4975x; 1.0009x over previous
import functools

import jax
import jax.numpy as jnp
from jax import lax
from jax.experimental import pallas as pl
from jax.experimental.pallas import tpu as pltpu
from jax.experimental.pallas import tpu_sc as plsc

N = 10000
E = 160000
D_IN = 256
D_H = 1024
D_OUT = 128

NCORES = 2
NTILES = 16
K = 80
EPAD = 163840
CHUNKS_PER_TILE = EPAD // (K * NTILES)
NPAD_ROWS = 48
SHARE_ROWS = 640
AGG_OUT_ROWS = NTILES * SHARE_ROWS
NP1 = AGG_OUT_ROWS + NPAD_ROWS
DEG_PAD = 10240
DEG_SHARE = DEG_PAD // NTILES

BN = 2000
NB_N = N // BN


HK = 128


def _hist_body(srcp, dstp, dego, degi, deg_sp, idx_v, ones_v, zer_v):
    cid = lax.axis_index("c")
    sid = lax.axis_index("s")
    for i in range(HK // 16):
        ones_v[pl.ds(i * 16, 16)] = jnp.ones((16,), jnp.float32)

    def zfill(i, c):
        zer_v[pl.ds(i * 16, 16)] = jnp.zeros((16,), jnp.float32)
        return c
    lax.fori_loop(0, DEG_SHARE // 16, zfill, 0)

    base = pl.multiple_of(sid * DEG_SHARE, 8)
    pltpu.sync_copy(zer_v, deg_sp.at[pl.ds(base, DEG_SHARE)])
    plsc.subcore_barrier()

    total_chunks = E // HK
    nbase, nrem = divmod(total_chunks, NTILES)
    n_j = nbase + jnp.where(sid < nrem, 1, 0)

    def accumulate(edge_ref):
        def body(j, c):
            e0 = pl.multiple_of((sid + NTILES * j) * HK, 8)
            pltpu.sync_copy(edge_ref.at[pl.ds(e0, HK)], idx_v)
            pltpu.sync_copy(ones_v, deg_sp.at[idx_v], add=True)
            return c
        lax.fori_loop(0, n_j, body, 0)

    @pl.when(cid == 0)
    def _():
        accumulate(srcp)

    @pl.when(cid == 1)
    def _():
        accumulate(dstp)

    plsc.subcore_barrier()

    @pl.when(cid == 0)
    def _():
        pltpu.sync_copy(deg_sp.at[pl.ds(base, DEG_SHARE)],
                        dego.at[pl.ds(base, DEG_SHARE)])

    @pl.when(cid == 1)
    def _():
        pltpu.sync_copy(deg_sp.at[pl.ds(base, DEG_SHARE)],
                        degi.at[pl.ds(base, DEG_SHARE)])


def _hist(src_p, dst_p):
    mesh = plsc.VectorSubcoreMesh(core_axis_name="c", subcore_axis_name="s")
    return pl.kernel(
        _hist_body,
        out_type=(pltpu.HBM((DEG_PAD,), jnp.float32),
                  pltpu.HBM((DEG_PAD,), jnp.float32)),
        mesh=mesh,
        scratch_types=[
            pltpu.VMEM_SHARED((DEG_PAD,), jnp.float32),
            pltpu.VMEM((HK,), jnp.int32),
            pltpu.VMEM((HK,), jnp.float32),
            pltpu.VMEM((DEG_SHARE,), jnp.float32),
        ],
    )(src_p, dst_p)


_EDGES_PER_TILE = EPAD // NTILES


def _unpack_chunk(pk_v, j, sidx, didx):
    for t in range(K // 16):
        pk = pk_v[pl.ds(j * K + t * 16, 16)]
        didx[pl.ds(t * 16, 16)] = lax.shift_right_logical(pk, 14)
        sidx[pl.ds(t * 16, 16)] = lax.bitwise_and(pk, (1 << 14) - 1)


def _agg_body(nblk_per_core, xb, packed, zeros_hbm, out,
              agg_sp, pk_v, rows, sidx, didx, gs, ss):
    cid = lax.axis_index("c")
    sid = lax.axis_index("s")
    nch = CHUNKS_PER_TILE

    ebase = pl.multiple_of(sid * _EDGES_PER_TILE, 8)
    pltpu.sync_copy(packed.at[pl.ds(ebase, _EDGES_PER_TILE)], pk_v)

    rbase = sid * SHARE_ROWS
    for b in range(nblk_per_core):
        bg = cid * nblk_per_core + b
        pltpu.sync_copy(zeros_hbm, agg_sp.at[pl.ds(rbase, SHARE_ROWS)])
        plsc.subcore_barrier()

        xrows = xb.at[bg]

        def prefetch(j1, q1):
            @pl.when(j1 >= 3)
            def _():
                pltpu.make_async_copy(rows[q1], agg_sp.at[didx[q1]],
                                      ss[q1]).wait()
            _unpack_chunk(pk_v, j1, sidx[q1], didx[q1])
            pltpu.async_copy(xrows.at[sidx[q1]], rows[q1], gs[q1])

        def phase(j, q):
            @pl.when(j < nch)
            def _():
                @pl.when(j + 1 < nch)
                def _():
                    prefetch(j + 1, (q + 1) % 3)
                pltpu.make_async_copy(xrows.at[sidx[q]], rows[q],
                                      gs[q]).wait()
                pltpu.async_copy(rows[q], agg_sp.at[didx[q]], ss[q], add=True)

        prefetch(0, 0)

        def body(p, c):
            j = 3 * p
            phase(j, 0)
            phase(j + 1, 1)
            phase(j + 2, 2)
            return c
        lax.fori_loop(0, (nch + 2) // 3, body, 0)
        for q in range(3):
            pltpu.make_async_copy(rows[q], agg_sp.at[didx[q]], ss[q]).wait()
        plsc.subcore_barrier()

        pltpu.sync_copy(agg_sp.at[pl.ds(rbase, SHARE_ROWS)],
                        out.at[pl.ds(rbase, SHARE_ROWS), pl.ds(bg * 128, 128)])
        plsc.subcore_barrier()


def _agg(xb, packed, nblocks, dtype=jnp.float32):
    mesh = plsc.VectorSubcoreMesh(core_axis_name="c", subcore_axis_name="s")

    def body(xb_r, packed_r, zeros_r, out_r, agg_sp, pk_v,
             r0, r1, r2, si0, si1, si2, di0, di1, di2,
             g0, g1, g2, s0, s1, s2):
        _agg_body(nblocks // NCORES, xb_r, packed_r, zeros_r, out_r,
                  agg_sp, pk_v, (r0, r1, r2), (si0, si1, si2),
                  (di0, di1, di2), (g0, g1, g2), (s0, s1, s2))

    return pl.kernel(
        body,
        out_type=pltpu.HBM((AGG_OUT_ROWS, nblocks * 128), dtype),
        mesh=mesh,
        scratch_types=[
            pltpu.VMEM_SHARED((NP1, 128), dtype),
            pltpu.VMEM((_EDGES_PER_TILE,), jnp.int32),
            pltpu.VMEM((K, 128), dtype),
            pltpu.VMEM((K, 128), dtype),
            pltpu.VMEM((K, 128), dtype),
            pltpu.VMEM((K,), jnp.int32),
            pltpu.VMEM((K,), jnp.int32),
            pltpu.VMEM((K,), jnp.int32),
            pltpu.VMEM((K,), jnp.int32),
            pltpu.VMEM((K,), jnp.int32),
            pltpu.VMEM((K,), jnp.int32),
            pltpu.SemaphoreType.DMA,
            pltpu.SemaphoreType.DMA,
            pltpu.SemaphoreType.DMA,
            pltpu.SemaphoreType.DMA,
            pltpu.SemaphoreType.DMA,
            pltpu.SemaphoreType.DMA,
        ],
    )(xb, packed, jnp.zeros((SHARE_ROWS, 128), dtype))


def _prep_body(x_ref, dego_ref, xs_ref):
    s = lax.rsqrt(jnp.maximum(dego_ref[...], 1.0))
    xs_ref[0] = x_ref[...] * s


def _prep(x, dego2):
    nb_f = D_IN // 128
    return pl.pallas_call(
        _prep_body,
        grid=(nb_f, NB_N),
        in_specs=[
            pl.BlockSpec((BN, 128), lambda b, n: (n, b)),
            pl.BlockSpec((BN, 1), lambda b, n: (n, 0)),
        ],
        out_specs=pl.BlockSpec((1, BN, 128), lambda b, n: (b, n, 0)),
        out_shape=jax.ShapeDtypeStruct((nb_f, N, 128), jnp.float32),
    )(x, dego2)


def _mm1_body(agg_ref, degi_ref, dego_ref, w_ref, b_ref, out_ref, lhs_ref):
    o = pl.program_id(1)

    @pl.when(o == 0)
    def _():
        lhs_ref[...] = agg_ref[...].astype(jnp.bfloat16)

    acc = jnp.dot(lhs_ref[...], w_ref[...], preferred_element_type=jnp.float32)
    s_in = lax.rsqrt(jnp.maximum(degi_ref[...], 1.0))
    s_out = lax.rsqrt(jnp.maximum(dego_ref[...], 1.0))
    out_ref[0] = jnp.maximum(acc * s_in + b_ref[...], 0.0) * s_out


def _mm1(agg1, degi2, dego2, W1, b1r):
    no = D_H // 128
    return pl.pallas_call(
        _mm1_body,
        grid=(NB_N, no),
        in_specs=[
            pl.BlockSpec((BN, D_IN), lambda n, o: (n, 0)),
            pl.BlockSpec((BN, 1), lambda n, o: (n, 0)),
            pl.BlockSpec((BN, 1), lambda n, o: (n, 0)),
            pl.BlockSpec((D_IN, 128), lambda n, o: (0, o)),
            pl.BlockSpec((1, 128), lambda n, o: (0, o)),
        ],
        out_specs=pl.BlockSpec((1, BN, 128), lambda n, o: (o, n, 0)),
        out_shape=jax.ShapeDtypeStruct((no, N, 128), jnp.float32),
        scratch_shapes=[pltpu.VMEM((BN, D_IN), jnp.bfloat16)],
    )(agg1, degi2, dego2, W1.astype(jnp.bfloat16), b1r)


def _mm2_body(agg_ref, degi_ref, w2_ref, b2_ref, wo_ref, bo_ref, out_ref,
              csum_ref, lhs_ref):
    n = pl.program_id(0)
    o = pl.program_id(1)
    no = pl.num_programs(1)

    @pl.when(o == 0)
    def _():
        lhs_ref[...] = agg_ref[...].astype(jnp.bfloat16)

    acc = jnp.dot(lhs_ref[...], w2_ref[...], preferred_element_type=jnp.float32)
    s_in = lax.rsqrt(jnp.maximum(degi_ref[...], 1.0))
    h = jnp.maximum(acc * s_in + b2_ref[...], 0.0)
    part = jnp.sum(h, axis=0, keepdims=True)

    for oi in range(no):
        @pl.when(jnp.logical_and(o == oi, n == 0))
        def _():
            csum_ref[pl.ds(oi, 1), :] = part

        @pl.when(jnp.logical_and(o == oi, n > 0))
        def _():
            csum_ref[pl.ds(oi, 1), :] += part

    @pl.when(n == NB_N - 1)
    def _():
        for oi in range(no):
            @pl.when(o == oi)
            def _():
                mean_o = csum_ref[pl.ds(oi, 1), :] * (1.0 / N)
                contrib = jnp.dot(mean_o, wo_ref[...],
                                  preferred_element_type=jnp.float32)

                @pl.when(o == 0)
                def _():
                    out_ref[...] = contrib + bo_ref[...]

                @pl.when(o > 0)
                def _():
                    out_ref[...] += contrib


def _mm2(agg2, degi2, W2, b2r, W_out, bor):
    no = D_H // 128
    return pl.pallas_call(
        _mm2_body,
        grid=(NB_N, no),
        in_specs=[
            pl.BlockSpec((BN, D_H), lambda n, o: (n, 0)),
            pl.BlockSpec((BN, 1), lambda n, o: (n, 0)),
            pl.BlockSpec((D_H, 128), lambda n, o: (0, o)),
            pl.BlockSpec((1, 128), lambda n, o: (0, o)),
            pl.BlockSpec((128, D_OUT), lambda n, o: (o, 0)),
            pl.BlockSpec((1, D_OUT), lambda n, o: (0, 0)),
        ],
        out_specs=pl.BlockSpec((1, D_OUT), lambda n, o: (0, 0)),
        out_shape=jax.ShapeDtypeStruct((1, D_OUT), jnp.float32),
        scratch_shapes=[pltpu.VMEM((no, 128), jnp.float32),
                        pltpu.VMEM((BN, D_H), jnp.bfloat16)],
    )(agg2, degi2, W2.astype(jnp.bfloat16), b2r, W_out, bor)


def kernel(x, edge_index, W1, b1, W2, b2, W_out, b_out):
    src = edge_index[0]
    dst = edge_index[1]
    pad_ids = jnp.arange(EPAD - E, dtype=jnp.int32) % NPAD_ROWS
    src_p = jnp.concatenate([src, pad_ids])
    dst_p = jnp.concatenate([dst, AGG_OUT_ROWS + pad_ids])
    packed = src_p + dst_p * (1 << 14)

    dego, degi = _hist(src, dst)
    dego2 = dego.reshape(DEG_PAD, 1)
    degi2 = degi.reshape(DEG_PAD, 1)

    xs = _prep(x, dego2)
    agg1 = _agg(xs, packed, D_IN // 128)
    h1s = _mm1(agg1, degi2, dego2, W1, b1.reshape(1, -1))
    agg2 = _agg(h1s, packed, D_H // 128)
    return _mm2(agg2, degi2, W2, b2.reshape(1, -1), W_out, b_out.reshape(1, -1))


# speedup vs baseline: 8.5633x; 1.0077x over previous
import functools

import jax
import jax.numpy as jnp
from jax import lax
from jax.experimental import pallas as pl
from jax.experimental.pallas import tpu as pltpu
from jax.experimental.pallas import tpu_sc as plsc

N = 10000
E = 160000
D_IN = 256
D_H = 1024
D_OUT = 128

NCORES = 2
NTILES = 16
K = 80
EPAD = 163840
CHUNKS_PER_TILE = EPAD // (K * NTILES)
NPAD_ROWS = 48
SHARE_ROWS = 640
AGG_OUT_ROWS = NTILES * SHARE_ROWS
NP1 = AGG_OUT_ROWS + NPAD_ROWS
DEG_PAD = 10240
DEG_SHARE = DEG_PAD // NTILES

BN = 2000
NB_N = N // BN


HK = 128


def _hist_body(srcp, dstp, dego, degi, deg_sp, idx_v, ones_v, zer_v):
    cid = lax.axis_index("c")
    sid = lax.axis_index("s")
    for i in range(HK // 16):
        ones_v[pl.ds(i * 16, 16)] = jnp.ones((16,), jnp.float32)

    def zfill(i, c):
        zer_v[pl.ds(i * 16, 16)] = jnp.zeros((16,), jnp.float32)
        return c
    lax.fori_loop(0, DEG_SHARE // 16, zfill, 0)

    base = pl.multiple_of(sid * DEG_SHARE, 8)
    pltpu.sync_copy(zer_v, deg_sp.at[pl.ds(base, DEG_SHARE)])
    plsc.subcore_barrier()

    total_chunks = E // HK
    nbase, nrem = divmod(total_chunks, NTILES)
    n_j = nbase + jnp.where(sid < nrem, 1, 0)

    def accumulate(edge_ref):
        def body(j, c):
            e0 = pl.multiple_of((sid + NTILES * j) * HK, 8)
            pltpu.sync_copy(edge_ref.at[pl.ds(e0, HK)], idx_v)
            pltpu.sync_copy(ones_v, deg_sp.at[idx_v], add=True)
            return c
        lax.fori_loop(0, n_j, body, 0)

    @pl.when(cid == 0)
    def _():
        accumulate(srcp)

    @pl.when(cid == 1)
    def _():
        accumulate(dstp)

    plsc.subcore_barrier()

    @pl.when(cid == 0)
    def _():
        pltpu.sync_copy(deg_sp.at[pl.ds(base, DEG_SHARE)],
                        dego.at[pl.ds(base, DEG_SHARE)])

    @pl.when(cid == 1)
    def _():
        pltpu.sync_copy(deg_sp.at[pl.ds(base, DEG_SHARE)],
                        degi.at[pl.ds(base, DEG_SHARE)])


def _hist(src_p, dst_p):
    mesh = plsc.VectorSubcoreMesh(core_axis_name="c", subcore_axis_name="s")
    return pl.kernel(
        _hist_body,
        out_type=(pltpu.HBM((DEG_PAD,), jnp.float32),
                  pltpu.HBM((DEG_PAD,), jnp.float32)),
        mesh=mesh,
        scratch_types=[
            pltpu.VMEM_SHARED((DEG_PAD,), jnp.float32),
            pltpu.VMEM((HK,), jnp.int32),
            pltpu.VMEM((HK,), jnp.float32),
            pltpu.VMEM((DEG_SHARE,), jnp.float32),
        ],
    )(src_p, dst_p)


_EDGES_PER_TILE = EPAD // NTILES


def _unpack_chunk(pk_v, j, sidx, didx):
    for t in range(K // 16):
        pk = pk_v[pl.ds(j * K + t * 16, 16)]
        didx[pl.ds(t * 16, 16)] = lax.shift_right_logical(pk, 14)
        sidx[pl.ds(t * 16, 16)] = lax.bitwise_and(pk, (1 << 14) - 1)


def _agg_body(nblk_per_core, xb, packed, zeros_hbm, out,
              agg_sp, pk_v, rows, sidx, didx, gs, ss):
    cid = lax.axis_index("c")
    sid = lax.axis_index("s")
    nch = CHUNKS_PER_TILE

    ebase = pl.multiple_of(sid * _EDGES_PER_TILE, 8)
    pltpu.sync_copy(packed.at[pl.ds(ebase, _EDGES_PER_TILE)], pk_v)

    rbase = sid * SHARE_ROWS
    for b in range(nblk_per_core):
        bg = cid * nblk_per_core + b
        pltpu.sync_copy(zeros_hbm, agg_sp.at[pl.ds(rbase, SHARE_ROWS)])
        plsc.subcore_barrier()

        xrows = xb.at[bg]

        def prefetch(j1, q1):
            @pl.when(j1 >= 3)
            def _():
                pltpu.make_async_copy(rows[q1], agg_sp.at[didx[q1]],
                                      ss[q1]).wait()
            _unpack_chunk(pk_v, j1, sidx[q1], didx[q1])
            pltpu.async_copy(xrows.at[sidx[q1]], rows[q1], gs[q1])

        def phase(j, q):
            @pl.when(j < nch)
            def _():
                @pl.when(j + 1 < nch)
                def _():
                    prefetch(j + 1, (q + 1) % 3)
                pltpu.make_async_copy(xrows.at[sidx[q]], rows[q],
                                      gs[q]).wait()
                pltpu.async_copy(rows[q], agg_sp.at[didx[q]], ss[q], add=True)

        prefetch(0, 0)

        def body(p, c):
            j = 3 * p
            phase(j, 0)
            phase(j + 1, 1)
            phase(j + 2, 2)
            return c
        lax.fori_loop(0, (nch + 2) // 3, body, 0)
        for q in range(3):
            pltpu.make_async_copy(rows[q], agg_sp.at[didx[q]], ss[q]).wait()
        plsc.subcore_barrier()

        pltpu.sync_copy(agg_sp.at[pl.ds(rbase, SHARE_ROWS)],
                        out.at[pl.ds(rbase, SHARE_ROWS), pl.ds(bg * 128, 128)])
        plsc.subcore_barrier()


def _agg(xb, packed, nblocks, dtype=jnp.float32):
    mesh = plsc.VectorSubcoreMesh(core_axis_name="c", subcore_axis_name="s")

    def body(xb_r, packed_r, zeros_r, out_r, agg_sp, pk_v,
             r0, r1, r2, si0, si1, si2, di0, di1, di2,
             g0, g1, g2, s0, s1, s2):
        _agg_body(nblocks // NCORES, xb_r, packed_r, zeros_r, out_r,
                  agg_sp, pk_v, (r0, r1, r2), (si0, si1, si2),
                  (di0, di1, di2), (g0, g1, g2), (s0, s1, s2))

    return pl.kernel(
        body,
        out_type=pltpu.HBM((AGG_OUT_ROWS, nblocks * 128), dtype),
        mesh=mesh,
        scratch_types=[
            pltpu.VMEM_SHARED((NP1, 128), dtype),
            pltpu.VMEM((_EDGES_PER_TILE,), jnp.int32),
            pltpu.VMEM((K, 128), dtype),
            pltpu.VMEM((K, 128), dtype),
            pltpu.VMEM((K, 128), dtype),
            pltpu.VMEM((K,), jnp.int32),
            pltpu.VMEM((K,), jnp.int32),
            pltpu.VMEM((K,), jnp.int32),
            pltpu.VMEM((K,), jnp.int32),
            pltpu.VMEM((K,), jnp.int32),
            pltpu.VMEM((K,), jnp.int32),
            pltpu.SemaphoreType.DMA,
            pltpu.SemaphoreType.DMA,
            pltpu.SemaphoreType.DMA,
            pltpu.SemaphoreType.DMA,
            pltpu.SemaphoreType.DMA,
            pltpu.SemaphoreType.DMA,
        ],
    )(xb, packed, jnp.zeros((SHARE_ROWS, 128), dtype))


def _prep_body(x_ref, dego_ref, xs_ref):
    s = lax.rsqrt(jnp.maximum(dego_ref[...], 1.0))
    xs_ref[0] = x_ref[...] * s


def _prep(x, dego2):
    nb_f = D_IN // 128
    return pl.pallas_call(
        _prep_body,
        grid=(nb_f, NB_N),
        in_specs=[
            pl.BlockSpec((BN, 128), lambda b, n: (n, b)),
            pl.BlockSpec((BN, 1), lambda b, n: (n, 0)),
        ],
        out_specs=pl.BlockSpec((1, BN, 128), lambda b, n: (b, n, 0)),
        out_shape=jax.ShapeDtypeStruct((nb_f, N, 128), jnp.float32),
    )(x, dego2)


def _mm1_body(agg_ref, degi_ref, dego_ref, w_ref, b_ref, out_ref, lhs_ref):
    o = pl.program_id(1)

    @pl.when(o == 0)
    def _():
        lhs_ref[...] = agg_ref[...].astype(jnp.bfloat16)

    acc = jnp.dot(lhs_ref[...], w_ref[...], preferred_element_type=jnp.float32)
    s_in = lax.rsqrt(jnp.maximum(degi_ref[...], 1.0))
    s_out = lax.rsqrt(jnp.maximum(dego_ref[...], 1.0))
    out_ref[0] = jnp.maximum(acc * s_in + b_ref[...], 0.0) * s_out


def _mm1(agg1, degi2, dego2, W1, b1r):
    no = W1.shape[1] // 128
    return pl.pallas_call(
        _mm1_body,
        grid=(NB_N, no),
        in_specs=[
            pl.BlockSpec((BN, D_IN), lambda n, o: (n, 0)),
            pl.BlockSpec((BN, 1), lambda n, o: (n, 0)),
            pl.BlockSpec((BN, 1), lambda n, o: (n, 0)),
            pl.BlockSpec((D_IN, 128), lambda n, o: (0, o)),
            pl.BlockSpec((1, 128), lambda n, o: (0, o)),
        ],
        out_specs=pl.BlockSpec((1, BN, 128), lambda n, o: (o, n, 0)),
        out_shape=jax.ShapeDtypeStruct((no, N, 128), jnp.float32),
        scratch_shapes=[pltpu.VMEM((BN, D_IN), jnp.bfloat16)],
    )(agg1, degi2, dego2, W1.astype(jnp.bfloat16), b1r)


def _mm2_body(aggA_ref, aggB_ref, degi_ref, w2a_ref, w2b_ref, b2_ref,
              wo_ref, bo_ref, out_ref, csum_ref, lhsA_ref, lhsB_ref):
    n = pl.program_id(0)
    o = pl.program_id(1)
    no = pl.num_programs(1)

    @pl.when(o == 0)
    def _():
        lhsA_ref[...] = aggA_ref[...].astype(jnp.bfloat16)
        lhsB_ref[...] = aggB_ref[...].astype(jnp.bfloat16)

    acc = (jnp.dot(lhsA_ref[...], w2a_ref[...],
                   preferred_element_type=jnp.float32) +
           jnp.dot(lhsB_ref[...], w2b_ref[...],
                   preferred_element_type=jnp.float32))
    s_in = lax.rsqrt(jnp.maximum(degi_ref[...], 1.0))
    h = jnp.maximum(acc * s_in + b2_ref[...], 0.0)
    part = jnp.sum(h, axis=0, keepdims=True)

    for oi in range(no):
        @pl.when(jnp.logical_and(o == oi, n == 0))
        def _():
            csum_ref[pl.ds(oi, 1), :] = part

        @pl.when(jnp.logical_and(o == oi, n > 0))
        def _():
            csum_ref[pl.ds(oi, 1), :] += part

    @pl.when(n == NB_N - 1)
    def _():
        for oi in range(no):
            @pl.when(o == oi)
            def _():
                mean_o = csum_ref[pl.ds(oi, 1), :] * (1.0 / N)
                contrib = jnp.dot(mean_o, wo_ref[...],
                                  preferred_element_type=jnp.float32)

                @pl.when(o == 0)
                def _():
                    out_ref[...] = contrib + bo_ref[...]

                @pl.when(o > 0)
                def _():
                    out_ref[...] += contrib


def _mm2(agg2a, agg2b, degi2, W2, b2r, W_out, bor):
    no = D_H // 128
    dh2 = D_H // 2
    w2bf = W2.astype(jnp.bfloat16)
    return pl.pallas_call(
        _mm2_body,
        grid=(NB_N, no),
        in_specs=[
            pl.BlockSpec((BN, dh2), lambda n, o: (n, 0)),
            pl.BlockSpec((BN, dh2), lambda n, o: (n, 0)),
            pl.BlockSpec((BN, 1), lambda n, o: (n, 0)),
            pl.BlockSpec((dh2, 128), lambda n, o: (0, o)),
            pl.BlockSpec((dh2, 128), lambda n, o: (0, o)),
            pl.BlockSpec((1, 128), lambda n, o: (0, o)),
            pl.BlockSpec((128, D_OUT), lambda n, o: (o, 0)),
            pl.BlockSpec((1, D_OUT), lambda n, o: (0, 0)),
        ],
        out_specs=pl.BlockSpec((1, D_OUT), lambda n, o: (0, 0)),
        out_shape=jax.ShapeDtypeStruct((1, D_OUT), jnp.float32),
        scratch_shapes=[pltpu.VMEM((no, 128), jnp.float32),
                        pltpu.VMEM((BN, dh2), jnp.bfloat16),
                        pltpu.VMEM((BN, dh2), jnp.bfloat16)],
    )(agg2a, agg2b, degi2, w2bf[:dh2], w2bf[dh2:], b2r, W_out, bor)


def kernel(x, edge_index, W1, b1, W2, b2, W_out, b_out):
    src = edge_index[0]
    dst = edge_index[1]
    pad_ids = jnp.arange(EPAD - E, dtype=jnp.int32) % NPAD_ROWS
    src_p = jnp.concatenate([src, pad_ids])
    dst_p = jnp.concatenate([dst, AGG_OUT_ROWS + pad_ids])
    packed = src_p + dst_p * (1 << 14)

    dego, degi = _hist(src, dst)
    dego2 = dego.reshape(DEG_PAD, 1)
    degi2 = degi.reshape(DEG_PAD, 1)

    xs = _prep(x, dego2)
    agg1 = _agg(xs, packed, D_IN // 128)
    dh2 = D_H // 2
    b1r = b1.reshape(1, -1)
    h1a = _mm1(agg1, degi2, dego2, W1[:, :dh2], b1r[:, :dh2])
    h1b = _mm1(agg1, degi2, dego2, W1[:, dh2:], b1r[:, dh2:])
    agg2a = _agg(h1a, packed, dh2 // 128)
    agg2b = _agg(h1b, packed, dh2 // 128)
    return _mm2(agg2a, agg2b, degi2, W2, b2.reshape(1, -1),
                W_out, b_out.reshape(1, -1))
